# baseline (speedup 1.0000x reference)
import functools

import jax
import jax.numpy as jnp
from jax import lax
from jax.experimental import pallas as pl
from jax.experimental.pallas import tpu as pltpu
from jax.experimental.pallas import tpu_sc as plsc

_N = 10000
_D = 128
_SD = 6
_E = 320000
_R = 272
_RP = 144
_NC = 2
_NS = 16
_NW = _NC * _NS
_K1 = 64
_K2 = 128
_EPW = 10112
_EPAD = _NW * _EPW
_NCH1 = _EPW // _K1
_NCH2 = _EPW // _K2
_ACC = 10240
_ACCPT = _ACC // _NS
_MR = 10048
_MRPT = _MR // _NS

_PERM = sum(([32 * j + 2 * k for k in range(16)]
             + [32 * j + 2 * k + 1 for k in range(16)]
             for j in range(4)), [])


def _pass1_body(a_hbm, b_hbm, srcg_hbm, srcs_hbm, tgtg_hbm, tgts_hbm,
                es_hbm, da_out, db_out, nb_out,
                abuf, bbuf, sgb, ssb, tgb, tsb, epb, eqb, esob,
                dash, dbsh, nbsh, zb, sem_tab, sem_idx, sem_s0, sem_s1):
    cid = lax.axis_index("c")
    sid = lax.axis_index("s")
    wid = sid * _NC + cid
    ebase = wid * _EPW
    iota16 = lax.iota(jnp.int32, 16)

    def _zb(i, _):
        zb[pl.ds(i * 16, 16)] = jnp.zeros((16,), jnp.float32)
        return ()
    lax.fori_loop(0, _ACCPT // 16, _zb, (), unroll=4)
    pltpu.sync_copy(zb, dash.at[pl.ds(sid * _ACCPT, _ACCPT)])
    pltpu.sync_copy(zb, dbsh.at[pl.ds(sid * _ACCPT, _ACCPT)])
    pltpu.sync_copy(zb, nbsh.at[pl.ds(sid * _ACCPT, _ACCPT)])
    plsc.subcore_barrier()

    def idx_start(g, sync=False):
        off = ebase + g * _K1
        slot = lax.rem(g, 3)
        pairs = ((srcg_hbm, sgb), (srcs_hbm, ssb), (tgtg_hbm, tgb),
                 (tgts_hbm, tsb))
        for hbm, buf in pairs:
            if sync:
                pltpu.sync_copy(hbm.at[pl.ds(off, _K1)], buf.at[slot])
            else:
                pltpu.async_copy(hbm.at[pl.ds(off, _K1)], buf.at[slot],
                                 sem_idx)

    def idx_wait(g):
        off = ebase + g * _K1
        slot = lax.rem(g, 3)
        for hbm, buf in ((srcg_hbm, sgb), (srcs_hbm, ssb), (tgtg_hbm, tgb),
                         (tgts_hbm, tsb)):
            pltpu.make_async_copy(hbm.at[pl.ds(off, _K1)], buf.at[slot],
                                  sem_idx).wait()

    def tab_start(g):
        slot = lax.rem(g, 2)
        islot = lax.rem(g, 3)
        pltpu.async_copy(a_hbm.at[sgb.at[islot]], abuf.at[slot], sem_tab)
        pltpu.async_copy(b_hbm.at[tgb.at[islot]], bbuf.at[slot], sem_tab)

    def tab_wait(g):
        slot = lax.rem(g, 2)
        islot = lax.rem(g, 3)
        pltpu.make_async_copy(a_hbm.at[sgb.at[islot]], abuf.at[slot],
                              sem_tab).wait()
        pltpu.make_async_copy(b_hbm.at[tgb.at[islot]], bbuf.at[slot],
                              sem_tab).wait()

    def scat_start(g):
        slot = lax.rem(g, 2)
        islot = lax.rem(g, 3)
        pltpu.sync_copy(esob.at[slot], dash.at[tsb.at[islot]], add=True)
        pltpu.sync_copy(epb.at[slot], dbsh.at[ssb.at[islot]], add=True)
        pltpu.sync_copy(eqb.at[slot], nbsh.at[ssb.at[islot]], add=True)
        pltpu.async_copy(esob.at[slot], es_hbm.at[pl.ds(ebase + g * _K1,
                                                        _K1)], sem_s0)

    def es_wait(g):
        slot = lax.rem(g, 2)
        pltpu.make_async_copy(esob.at[slot],
                              es_hbm.at[pl.ds(ebase + g * _K1, _K1)],
                              sem_s0).wait()

    idx_start(0, sync=True)
    tab_start(0)
    idx_start(1)

    def gbody(g, _):
        slot = lax.rem(g, 2)
        tab_wait(g)

        @pl.when(g < _NCH1 - 1)
        def _():
            idx_wait(g + 1)
            tab_start(g + 1)

        @pl.when(g < _NCH1 - 2)
        def _():
            idx_start(g + 2)

        @pl.when(g >= 2)
        def _():
            es_wait(g - 2)

        a2 = abuf.at[slot]
        b2 = bbuf.at[slot]
        hmask = jnp.full((16,), -65536, jnp.int32)
        smask = jnp.where(iota16 < 3, jnp.int32(-1), jnp.int32(0))
        for grp in range(_K1 // 16):
            def ebody(i, carry):
                sv, pv, qvv = carry
                e = grp * 16 + i
                accs = jnp.zeros((16,), jnp.float32)
                accp = jnp.zeros((16,), jnp.float32)
                qs = jnp.float32(0)
                for k in range(9):
                    xa = a2[e, pl.ds(k * 16, 16)]
                    xb = b2[e, pl.ds(k * 16, 16)]
                    if k == 8:
                        qs = plsc.bitcast(xb, jnp.float32)[3]
                        xb = xb & smask
                    la = plsc.bitcast(xa << 16, jnp.float32)
                    ha = plsc.bitcast(xa & hmask, jnp.float32)
                    lb = plsc.bitcast(xb << 16, jnp.float32)
                    hb = plsc.bitcast(xb & hmask, jnp.float32)
                    prod = la * lb + ha * hb
                    if k < 4 or k == 8:
                        accs = accs + prod
                    else:
                        accp = accp + prod
                m = (iota16 == i).astype(jnp.float32)
                sv = sv + jnp.full((16,), jnp.sum(accs), jnp.float32) * m
                pv = pv + jnp.full((16,), jnp.sum(accp), jnp.float32) * m
                qvv = qvv + jnp.full((16,), qs, jnp.float32) * m
                return sv, pv, qvv
            z16 = jnp.zeros((16,), jnp.float32)
            sv, pv, qvv = lax.fori_loop(0, 16, ebody, (z16, z16, z16),
                                        unroll=2)
            es = jnp.exp(sv)
            ep = jnp.exp(pv)
            esob[slot, pl.ds(grp * 16, 16)] = es
            epb[slot, pl.ds(grp * 16, 16)] = ep
            eqb[slot, pl.ds(grp * 16, 16)] = ep * qvv

        scat_start(g)
        return ()

    lax.fori_loop(0, _NCH1, gbody, ())
    es_wait(_NCH1 - 2)
    es_wait(_NCH1 - 1)
    plsc.subcore_barrier()
    pltpu.sync_copy(dash.at[pl.ds(sid * _ACCPT, _ACCPT)],
                    da_out.at[cid, pl.ds(sid * _ACCPT, _ACCPT)])
    pltpu.sync_copy(dbsh.at[pl.ds(sid * _ACCPT, _ACCPT)],
                    db_out.at[cid, pl.ds(sid * _ACCPT, _ACCPT)])
    pltpu.sync_copy(nbsh.at[pl.ds(sid * _ACCPT, _ACCPT)],
                    nb_out.at[cid, pl.ds(sid * _ACCPT, _ACCPT)])


@functools.cache
def _make_pass1():
  return pl.kernel(
    _pass1_body,
    out_type=(jax.ShapeDtypeStruct((_EPAD,), jnp.float32),
              jax.ShapeDtypeStruct((_NC, _ACC), jnp.float32),
              jax.ShapeDtypeStruct((_NC, _ACC), jnp.float32),
              jax.ShapeDtypeStruct((_NC, _ACC), jnp.float32)),
    mesh=plsc.VectorSubcoreMesh(core_axis_name="c", subcore_axis_name="s"),
    compiler_params=pltpu.CompilerParams(use_tc_tiling_on_sc=False,
                                         needs_layout_passes=False),
    scratch_types=(
        pltpu.VMEM((2, _K1, _RP), jnp.int32),
        pltpu.VMEM((2, _K1, _RP), jnp.int32),
        pltpu.VMEM((3, _K1), jnp.int32),
        pltpu.VMEM((3, _K1), jnp.int32),
        pltpu.VMEM((3, _K1), jnp.int32),
        pltpu.VMEM((3, _K1), jnp.int32),
        pltpu.VMEM((2, _K1), jnp.float32),
        pltpu.VMEM((2, _K1), jnp.float32),
        pltpu.VMEM((2, _K1), jnp.float32),
        pltpu.VMEM_SHARED((_ACC,), jnp.float32),
        pltpu.VMEM_SHARED((_ACC,), jnp.float32),
        pltpu.VMEM_SHARED((_ACC,), jnp.float32),
        pltpu.VMEM((_ACCPT,), jnp.float32),
        pltpu.SemaphoreType.DMA,
        pltpu.SemaphoreType.DMA,
        pltpu.SemaphoreType.DMA,
        pltpu.SemaphoreType.DMA,
    ),
  )


def _pass2_body(g_hbm, srcg_hbm, tgts_hbm, es_hbm,
                macc_out,
                gibuf, rbuf, sgb, tsb, esb, msh,
                sem_tab, sem_idx):
    cid = lax.axis_index("c")
    sid = lax.axis_index("s")
    wid = sid * _NC + cid
    ebase = wid * _EPW
    def _zg(r, _):
        for j in range(_D // 16):
            rbuf[r, pl.ds(j * 16, 16)] = jnp.zeros((16,), jnp.float32)
        return ()
    lax.fori_loop(0, _K2, _zg, (), unroll=2)
    for kk in range(4):
        pltpu.sync_copy(rbuf, msh.at[pl.ds(sid * _MRPT + kk * _K2, _K2)])
    pltpu.sync_copy(rbuf.at[pl.ds(0, _MRPT - 4 * _K2)],
                    msh.at[pl.ds(sid * _MRPT + 4 * _K2, _MRPT - 4 * _K2)])
    plsc.subcore_barrier()

    def idx_start(g, sync=False):
        off = ebase + g * _K2
        slot = lax.rem(g, 3)
        for hbm, buf in ((srcg_hbm, sgb), (tgts_hbm, tsb)):
            if sync:
                pltpu.sync_copy(hbm.at[pl.ds(off, _K2)], buf.at[slot])
            else:
                pltpu.async_copy(hbm.at[pl.ds(off, _K2)], buf.at[slot],
                                 sem_idx)

    def idx_wait(g):
        off = ebase + g * _K2
        slot = lax.rem(g, 3)
        for hbm, buf in ((srcg_hbm, sgb), (tgts_hbm, tsb)):
            pltpu.make_async_copy(hbm.at[pl.ds(off, _K2)], buf.at[slot],
                                  sem_idx).wait()

    def tab_start(g):
        slot = lax.rem(g, 3)
        off = ebase + g * _K2
        pltpu.async_copy(g_hbm.at[sgb.at[slot]], gibuf.at[lax.rem(g, 2)],
                         sem_tab)
        pltpu.async_copy(es_hbm.at[pl.ds(off, _K2)], esb.at[lax.rem(g, 2)],
                         sem_tab)

    def tab_wait(g):
        slot = lax.rem(g, 3)
        off = ebase + g * _K2
        pltpu.make_async_copy(g_hbm.at[sgb.at[slot]],
                              gibuf.at[lax.rem(g, 2)], sem_tab).wait()
        pltpu.make_async_copy(es_hbm.at[pl.ds(off, _K2)],
                              esb.at[lax.rem(g, 2)], sem_tab).wait()

    idx_start(0, sync=True)
    tab_start(0)
    idx_start(1)

    hmask = jnp.full((16,), -65536, jnp.int32)

    def gbody(g, _):
        slot = lax.rem(g, 3)
        eslot = lax.rem(g, 2)
        tab_wait(g)

        @pl.when(g < _NCH2 - 1)
        def _():
            idx_wait(g + 1)
            tab_start(g + 1)

        @pl.when(g < _NCH2 - 2)
        def _():
            idx_start(g + 2)

        def egrp(gr, _):
            esv = esb[eslot, pl.ds(gr * 16, 16)]
            base = gr * 16
            for j16 in range(16):
                sc = jnp.full((16,), esv[j16], jnp.float32)
                e = base + j16
                for j in range(_D // 32):
                    x = gibuf[eslot, e, pl.ds(j * 16, 16)]
                    lo = plsc.bitcast(x << 16, jnp.float32)
                    hi = plsc.bitcast(x & hmask, jnp.float32)
                    rbuf[e, pl.ds(j * 32, 16)] = lo * sc
                    rbuf[e, pl.ds(j * 32 + 16, 16)] = hi * sc
            return ()
        lax.fori_loop(0, _K2 // 16, egrp, ())

        pltpu.sync_copy(rbuf, msh.at[tsb.at[slot]], add=True)
        return ()

    lax.fori_loop(0, _NCH2, gbody, ())
    plsc.subcore_barrier()
    pltpu.sync_copy(msh.at[pl.ds(sid * _MRPT, _MRPT)],
                    macc_out.at[cid, pl.ds(sid * _MRPT, _MRPT)])


@functools.cache
def _make_pass2():
  return pl.kernel(
    _pass2_body,
    out_type=jax.ShapeDtypeStruct((_NC, _MR, _D), jnp.float32),
    mesh=plsc.VectorSubcoreMesh(core_axis_name="c", subcore_axis_name="s"),
    compiler_params=pltpu.CompilerParams(use_tc_tiling_on_sc=False,
                                         needs_layout_passes=False),
    scratch_types=(
        pltpu.VMEM((2, _K2, _D // 2), jnp.int32),
        pltpu.VMEM((_K2, _D), jnp.float32),
        pltpu.VMEM((3, _K2), jnp.int32),
        pltpu.VMEM((3, _K2), jnp.int32),
        pltpu.VMEM((2, _K2), jnp.float32),
        pltpu.VMEM_SHARED((_MR, _D), jnp.float32),
        pltpu.SemaphoreType.DMA,
        pltpu.SemaphoreType.DMA,
    ),
  )


_BLK = 2000


def _tc1_body(h_ref, w_ref, b_ref, a_ref, bt_ref, hphi_ref, q_ref):
    hb = h_ref[...]
    t = jnp.dot(hb, w_ref[...],
                preferred_element_type=jnp.float32) + b_ref[...]
    s = hb[:, :_SD]
    q_ref[...] = jnp.exp(-t[:, 512:513])
    z10 = jnp.zeros((_BLK, 10), jnp.float32)
    a_ref[...] = jnp.concatenate([t[:, 0:256], s, z10],
                                 axis=1).astype(jnp.bfloat16)
    bt_ref[...] = jnp.concatenate([hb, t[:, 256:384], s, z10],
                                  axis=1).astype(jnp.bfloat16)
    hphi_ref[...] = t[:, 384:512]


_tc1 = pl.pallas_call(
    _tc1_body,
    grid=(_N // _BLK,),
    in_specs=[
        pl.BlockSpec((_BLK, _D), lambda i: (i, 0)),
        pl.BlockSpec((_D, 513), lambda i: (0, 0)),
        pl.BlockSpec((1, 513), lambda i: (0, 0)),
    ],
    out_specs=[
        pl.BlockSpec((_BLK, _R), lambda i: (i, 0)),
        pl.BlockSpec((_BLK, _R), lambda i: (i, 0)),
        pl.BlockSpec((_BLK, _D), lambda i: (i, 0)),
        pl.BlockSpec((_BLK, 1), lambda i: (i, 0)),
    ],
    out_shape=[
        jax.ShapeDtypeStruct((_N, _R), jnp.bfloat16),
        jax.ShapeDtypeStruct((_N, _R), jnp.bfloat16),
        jax.ShapeDtypeStruct((_N, _D), jnp.float32),
        jax.ShapeDtypeStruct((_N, 1), jnp.float32),
    ],
)


def _tc2_body(hphi_ref, da0, da1, db0, db1, nb0, nb1, g_ref, v_ref):
    da = da0[...] + da1[...]
    db = db0[...] + db1[...]
    nb = nb0[...] + nb1[...]
    st = nb / (db + 1e-16)
    dd = -jnp.log(st + 1e-8)
    rho = 1.0 / (1.0 + jnp.exp(-(dd - 0.5)))
    u = 1.0 - rho
    v_ref[...] = 1.0 / (da + 1e-16)
    g_ref[...] = (u * hphi_ref[...]).astype(jnp.bfloat16)


_tc2 = pl.pallas_call(
    _tc2_body,
    grid=(_N // _BLK,),
    in_specs=[pl.BlockSpec((_BLK, _D), lambda i: (i, 0))]
    + [pl.BlockSpec((_BLK, 1), lambda i: (i, 0))] * 6,
    out_specs=[
        pl.BlockSpec((_BLK, _D), lambda i: (i, 0)),
        pl.BlockSpec((_BLK, 1), lambda i: (i, 0)),
    ],
    out_shape=[
        jax.ShapeDtypeStruct((_N, _D), jnp.bfloat16),
        jax.ShapeDtypeStruct((_N, 1), jnp.float32),
    ],
)


def _tc3_body(h_ref, macc0_ref, macc1_ref, v_ref, wself_ref, wa_ref,
              wstr_ref, bias_ref, lng_ref, lnb_ref, out_ref):
    hb = h_ref[...]
    m_att = v_ref[...] * (macc0_ref[0] + macc1_ref[0])
    s = hb[:, :_SD]
    pre = (jnp.dot(hb, wself_ref[...], preferred_element_type=jnp.float32)
           + jnp.dot(m_att, wa_ref[...], preferred_element_type=jnp.float32)
           + jnp.dot(s, wstr_ref[...], preferred_element_type=jnp.float32)
           + bias_ref[...])
    hn = jnp.maximum(pre, 0.0) + hb
    mu = jnp.mean(hn, axis=1, keepdims=True)
    var = jnp.mean((hn - mu) ** 2, axis=1, keepdims=True)
    out_ref[...] = ((hn - mu) * lax.rsqrt(var + 1e-5) * lng_ref[...]
                    + lnb_ref[...])


_tc3 = pl.pallas_call(
    _tc3_body,
    grid=(_N // _BLK,),
    in_specs=[
        pl.BlockSpec((_BLK, _D), lambda i: (i, 0)),
        pl.BlockSpec((1, _BLK, _D), lambda i: (0, i, 0)),
        pl.BlockSpec((1, _BLK, _D), lambda i: (1, i, 0)),
        pl.BlockSpec((_BLK, 1), lambda i: (i, 0)),
        pl.BlockSpec((_D, _D), lambda i: (0, 0)),
        pl.BlockSpec((_D, _D), lambda i: (0, 0)),
        pl.BlockSpec((_SD, _D), lambda i: (0, 0)),
        pl.BlockSpec((1, _D), lambda i: (0, 0)),
        pl.BlockSpec((1, _D), lambda i: (0, 0)),
        pl.BlockSpec((1, _D), lambda i: (0, 0)),
    ],
    out_specs=pl.BlockSpec((_BLK, _D), lambda i: (i, 0)),
    out_shape=jax.ShapeDtypeStruct((_N, _D), jnp.float32),
)


def kernel(h, edge_index, W_att, phi_w, phi_b, W_p, W_pp, fdef_w, fdef_b,
           Wself_w, Wself_b, WA_w, WA_b, Wstr_w, Wstr_b, ln_g, ln_b):
    f32 = jnp.float32
    wcat = jnp.concatenate([W_att, W_p, W_pp, phi_w, fdef_w], axis=1)
    bcat = jnp.concatenate([jnp.zeros((384,), f32), phi_b, fdef_b])[None, :]
    a_tab, b_tab, hphi, qv = _tc1(h, wcat, bcat)

    a_i = lax.bitcast_convert_type(a_tab.reshape(_N, _R // 2, 2), jnp.int32)
    b_i = lax.bitcast_convert_type(b_tab.reshape(_N, _R // 2, 2), jnp.int32)
    q_i = lax.bitcast_convert_type(qv, jnp.int32)
    a_pk = jnp.concatenate([a_i, jnp.zeros((_N, _RP - _R // 2), jnp.int32)],
                           axis=1)
    b_pk = jnp.concatenate([b_i[:, :131], q_i,
                            jnp.zeros((_N, 12), jnp.int32)], axis=1)

    src = edge_index[0]
    tgt = edge_index[1]
    iar = jnp.arange(_EPAD - _E, dtype=jnp.int32) % 32
    src_g = jnp.concatenate([src, iar])
    tgt_g = jnp.concatenate([tgt, iar])
    src_s = jnp.concatenate([src, _N + iar])
    tgt_s = jnp.concatenate([tgt, _N + iar])

    es, da, db, nb = _make_pass1()(a_pk, b_pk, src_g, src_s, tgt_g, tgt_s)

    gt, vp = _tc2(hphi,
                  da[0][:, None], da[1][:, None],
                  db[0][:, None], db[1][:, None],
                  nb[0][:, None], nb[1][:, None])

    gt_i32 = lax.bitcast_convert_type(gt.reshape(_N, _D // 2, 2),
                                      jnp.int32)
    macc = _make_pass2()(gt_i32, src_g, tgt_s, es)

    wa_perm = WA_w[_PERM, :]

    out = _tc3(h, macc, macc, vp,
               Wself_w, wa_perm, Wstr_w,
               (Wself_b + WA_b + Wstr_b)[None, :],
               ln_g[None, :], ln_b[None, :])
    return out


# speedup vs baseline: 1.3265x; 1.3265x over previous
import functools

import jax
import jax.numpy as jnp
from jax import lax
from jax.experimental import pallas as pl
from jax.experimental.pallas import tpu as pltpu
from jax.experimental.pallas import tpu_sc as plsc

_N = 10000
_D = 128
_SD = 6
_E = 320000
_R = 272
_RP = 144
_NC = 2
_NS = 16
_NW = _NC * _NS
_K1 = 64
_K2 = 128
_EPW = 10112
_EPAD = _NW * _EPW
_NCH1 = _EPW // _K1
_NCH2 = _EPW // _K2
_ACC = 10240
_ACCPT = _ACC // _NS
_MR = 10048
_MRPT = _MR // _NS

_PERM = sum(([32 * j + 2 * k for k in range(16)]
             + [32 * j + 2 * k + 1 for k in range(16)]
             for j in range(4)), [])


def _pass1_body(a_hbm, b_hbm, srcg_hbm, srcs_hbm, tgtg_hbm, tgts_hbm,
                es_hbm, da_out, db_out, nb_out,
                abuf, bbuf, sgb, ssb, tgb, tsb, epb, eqb, esob,
                dash, dbsh, nbsh, zb, sem_tab, sem_idx, sem_s0, sem_s1):
    cid = lax.axis_index("c")
    sid = lax.axis_index("s")
    wid = sid * _NC + cid
    ebase = wid * _EPW
    iota16 = lax.iota(jnp.int32, 16)

    def _zb(i, _):
        zb[pl.ds(i * 16, 16)] = jnp.zeros((16,), jnp.float32)
        return ()
    lax.fori_loop(0, _ACCPT // 16, _zb, (), unroll=4)
    pltpu.sync_copy(zb, dash.at[pl.ds(sid * _ACCPT, _ACCPT)])
    pltpu.sync_copy(zb, dbsh.at[pl.ds(sid * _ACCPT, _ACCPT)])
    pltpu.sync_copy(zb, nbsh.at[pl.ds(sid * _ACCPT, _ACCPT)])
    plsc.subcore_barrier()

    def idx_start(g, sync=False):
        off = ebase + g * _K1
        slot = lax.rem(g, 3)
        pairs = ((srcg_hbm, sgb), (srcs_hbm, ssb), (tgtg_hbm, tgb),
                 (tgts_hbm, tsb))
        for hbm, buf in pairs:
            if sync:
                pltpu.sync_copy(hbm.at[pl.ds(off, _K1)], buf.at[slot])
            else:
                pltpu.async_copy(hbm.at[pl.ds(off, _K1)], buf.at[slot],
                                 sem_idx)

    def idx_wait(g):
        off = ebase + g * _K1
        slot = lax.rem(g, 3)
        for hbm, buf in ((srcg_hbm, sgb), (srcs_hbm, ssb), (tgtg_hbm, tgb),
                         (tgts_hbm, tsb)):
            pltpu.make_async_copy(hbm.at[pl.ds(off, _K1)], buf.at[slot],
                                  sem_idx).wait()

    def tab_start(g):
        slot = lax.rem(g, 2)
        islot = lax.rem(g, 3)
        pltpu.async_copy(a_hbm.at[sgb.at[islot]], abuf.at[slot], sem_tab)
        pltpu.async_copy(b_hbm.at[tgb.at[islot]], bbuf.at[slot], sem_tab)

    def tab_wait(g):
        slot = lax.rem(g, 2)
        islot = lax.rem(g, 3)
        pltpu.make_async_copy(a_hbm.at[sgb.at[islot]], abuf.at[slot],
                              sem_tab).wait()
        pltpu.make_async_copy(b_hbm.at[tgb.at[islot]], bbuf.at[slot],
                              sem_tab).wait()

    def scat_start(g):
        slot = lax.rem(g, 2)
        islot = lax.rem(g, 3)
        pltpu.sync_copy(esob.at[slot], dash.at[tsb.at[islot]], add=True)
        pltpu.sync_copy(epb.at[slot], dbsh.at[ssb.at[islot]], add=True)
        pltpu.sync_copy(eqb.at[slot], nbsh.at[ssb.at[islot]], add=True)
        pltpu.async_copy(esob.at[slot], es_hbm.at[pl.ds(ebase + g * _K1,
                                                        _K1)], sem_s0)

    def es_wait(g):
        slot = lax.rem(g, 2)
        pltpu.make_async_copy(esob.at[slot],
                              es_hbm.at[pl.ds(ebase + g * _K1, _K1)],
                              sem_s0).wait()

    idx_start(0, sync=True)
    tab_start(0)
    idx_start(1)

    def gbody(g, _):
        slot = lax.rem(g, 2)
        tab_wait(g)

        @pl.when(g < _NCH1 - 1)
        def _():
            idx_wait(g + 1)
            tab_start(g + 1)

        @pl.when(g < _NCH1 - 2)
        def _():
            idx_start(g + 2)

        @pl.when(g >= 2)
        def _():
            es_wait(g - 2)

        a2 = abuf.at[slot]
        b2 = bbuf.at[slot]
        hmask = jnp.full((16,), -65536, jnp.int32)
        for grp in range(_K1 // 16):
            def ebody(i, carry):
                sv, pv, qvv = carry
                e = grp * 16 + i
                accs = jnp.zeros((16,), jnp.float32)
                accp = jnp.zeros((16,), jnp.float32)
                qs = jnp.float32(0)
                for k in range(9):
                    xa = a2[e, pl.ds(k * 16, 16)]
                    xb = b2[e, pl.ds(k * 16, 16)]
                    la = plsc.bitcast(xa << 16, jnp.float32)
                    ha = plsc.bitcast(xa & hmask, jnp.float32)
                    lb = plsc.bitcast(xb << 16, jnp.float32)
                    hb = plsc.bitcast(xb & hmask, jnp.float32)
                    prod = la * lb + ha * hb
                    if k < 4 or k == 8:
                        accs = accs + prod
                    else:
                        accp = accp + prod
                    if k == 8:
                        qs = hb[0]
                m = (iota16 == i).astype(jnp.float32)
                sv = sv + jnp.full((16,), jnp.sum(accs), jnp.float32) * m
                pv = pv + jnp.full((16,), jnp.sum(accp), jnp.float32) * m
                qvv = qvv + jnp.full((16,), qs, jnp.float32) * m
                return sv, pv, qvv
            z16 = jnp.zeros((16,), jnp.float32)
            sv, pv, qvv = lax.fori_loop(0, 16, ebody, (z16, z16, z16),
                                        unroll=2)
            es = jnp.exp(sv)
            ep = jnp.exp(pv)
            esob[slot, pl.ds(grp * 16, 16)] = es
            epb[slot, pl.ds(grp * 16, 16)] = ep
            eqb[slot, pl.ds(grp * 16, 16)] = ep * qvv

        scat_start(g)
        return ()

    lax.fori_loop(0, _NCH1, gbody, ())
    es_wait(_NCH1 - 2)
    es_wait(_NCH1 - 1)
    plsc.subcore_barrier()
    pltpu.sync_copy(dash.at[pl.ds(sid * _ACCPT, _ACCPT)],
                    da_out.at[cid, pl.ds(sid * _ACCPT, _ACCPT)])
    pltpu.sync_copy(dbsh.at[pl.ds(sid * _ACCPT, _ACCPT)],
                    db_out.at[cid, pl.ds(sid * _ACCPT, _ACCPT)])
    pltpu.sync_copy(nbsh.at[pl.ds(sid * _ACCPT, _ACCPT)],
                    nb_out.at[cid, pl.ds(sid * _ACCPT, _ACCPT)])


@functools.cache
def _make_pass1():
  return pl.kernel(
    _pass1_body,
    out_type=(jax.ShapeDtypeStruct((_EPAD,), jnp.float32),
              jax.ShapeDtypeStruct((_NC, _ACC), jnp.float32),
              jax.ShapeDtypeStruct((_NC, _ACC), jnp.float32),
              jax.ShapeDtypeStruct((_NC, _ACC), jnp.float32)),
    mesh=plsc.VectorSubcoreMesh(core_axis_name="c", subcore_axis_name="s"),
    compiler_params=pltpu.CompilerParams(use_tc_tiling_on_sc=False,
                                         needs_layout_passes=False),
    scratch_types=(
        pltpu.VMEM((2, _K1, _RP), jnp.int32),
        pltpu.VMEM((2, _K1, _RP), jnp.int32),
        pltpu.VMEM((3, _K1), jnp.int32),
        pltpu.VMEM((3, _K1), jnp.int32),
        pltpu.VMEM((3, _K1), jnp.int32),
        pltpu.VMEM((3, _K1), jnp.int32),
        pltpu.VMEM((2, _K1), jnp.float32),
        pltpu.VMEM((2, _K1), jnp.float32),
        pltpu.VMEM((2, _K1), jnp.float32),
        pltpu.VMEM_SHARED((_ACC,), jnp.float32),
        pltpu.VMEM_SHARED((_ACC,), jnp.float32),
        pltpu.VMEM_SHARED((_ACC,), jnp.float32),
        pltpu.VMEM((_ACCPT,), jnp.float32),
        pltpu.SemaphoreType.DMA,
        pltpu.SemaphoreType.DMA,
        pltpu.SemaphoreType.DMA,
        pltpu.SemaphoreType.DMA,
    ),
  )


def _pass2_body(g_hbm, srcg_hbm, tgts_hbm, es_hbm,
                macc_out,
                gibuf, rbuf, sgb, tsb, esb, msh,
                sem_tab, sem_idx):
    cid = lax.axis_index("c")
    sid = lax.axis_index("s")
    wid = sid * _NC + cid
    ebase = wid * _EPW
    def _zg(r, _):
        for j in range(_D // 16):
            rbuf[r, pl.ds(j * 16, 16)] = jnp.zeros((16,), jnp.float32)
        return ()
    lax.fori_loop(0, _K2, _zg, (), unroll=2)
    for kk in range(4):
        pltpu.sync_copy(rbuf, msh.at[pl.ds(sid * _MRPT + kk * _K2, _K2)])
    pltpu.sync_copy(rbuf.at[pl.ds(0, _MRPT - 4 * _K2)],
                    msh.at[pl.ds(sid * _MRPT + 4 * _K2, _MRPT - 4 * _K2)])
    plsc.subcore_barrier()

    def idx_start(g, sync=False):
        off = ebase + g * _K2
        slot = lax.rem(g, 3)
        for hbm, buf in ((srcg_hbm, sgb), (tgts_hbm, tsb)):
            if sync:
                pltpu.sync_copy(hbm.at[pl.ds(off, _K2)], buf.at[slot])
            else:
                pltpu.async_copy(hbm.at[pl.ds(off, _K2)], buf.at[slot],
                                 sem_idx)

    def idx_wait(g):
        off = ebase + g * _K2
        slot = lax.rem(g, 3)
        for hbm, buf in ((srcg_hbm, sgb), (tgts_hbm, tsb)):
            pltpu.make_async_copy(hbm.at[pl.ds(off, _K2)], buf.at[slot],
                                  sem_idx).wait()

    def tab_start(g):
        slot = lax.rem(g, 3)
        off = ebase + g * _K2
        pltpu.async_copy(g_hbm.at[sgb.at[slot]], gibuf.at[lax.rem(g, 2)],
                         sem_tab)
        pltpu.async_copy(es_hbm.at[pl.ds(off, _K2)], esb.at[lax.rem(g, 2)],
                         sem_tab)

    def tab_wait(g):
        slot = lax.rem(g, 3)
        off = ebase + g * _K2
        pltpu.make_async_copy(g_hbm.at[sgb.at[slot]],
                              gibuf.at[lax.rem(g, 2)], sem_tab).wait()
        pltpu.make_async_copy(es_hbm.at[pl.ds(off, _K2)],
                              esb.at[lax.rem(g, 2)], sem_tab).wait()

    idx_start(0, sync=True)
    tab_start(0)
    idx_start(1)

    hmask = jnp.full((16,), -65536, jnp.int32)

    def gbody(g, _):
        slot = lax.rem(g, 3)
        eslot = lax.rem(g, 2)
        tab_wait(g)

        @pl.when(g < _NCH2 - 1)
        def _():
            idx_wait(g + 1)
            tab_start(g + 1)

        @pl.when(g < _NCH2 - 2)
        def _():
            idx_start(g + 2)

        def egrp(gr, _):
            esv = esb[eslot, pl.ds(gr * 16, 16)]
            base = gr * 16
            for j16 in range(16):
                sc = jnp.full((16,), esv[j16], jnp.float32)
                e = base + j16
                for j in range(_D // 32):
                    x = gibuf[eslot, e, pl.ds(j * 16, 16)]
                    lo = plsc.bitcast(x << 16, jnp.float32)
                    hi = plsc.bitcast(x & hmask, jnp.float32)
                    rbuf[e, pl.ds(j * 32, 16)] = lo * sc
                    rbuf[e, pl.ds(j * 32 + 16, 16)] = hi * sc
            return ()
        lax.fori_loop(0, _K2 // 16, egrp, ())

        pltpu.sync_copy(rbuf, msh.at[tsb.at[slot]], add=True)
        return ()

    lax.fori_loop(0, _NCH2, gbody, ())
    plsc.subcore_barrier()
    pltpu.sync_copy(msh.at[pl.ds(sid * _MRPT, _MRPT)],
                    macc_out.at[cid, pl.ds(sid * _MRPT, _MRPT)])


@functools.cache
def _make_pass2():
  return pl.kernel(
    _pass2_body,
    out_type=jax.ShapeDtypeStruct((_NC, _MR, _D), jnp.float32),
    mesh=plsc.VectorSubcoreMesh(core_axis_name="c", subcore_axis_name="s"),
    compiler_params=pltpu.CompilerParams(use_tc_tiling_on_sc=False,
                                         needs_layout_passes=False),
    scratch_types=(
        pltpu.VMEM((2, _K2, _D // 2), jnp.int32),
        pltpu.VMEM((_K2, _D), jnp.float32),
        pltpu.VMEM((3, _K2), jnp.int32),
        pltpu.VMEM((3, _K2), jnp.int32),
        pltpu.VMEM((2, _K2), jnp.float32),
        pltpu.VMEM_SHARED((_MR, _D), jnp.float32),
        pltpu.SemaphoreType.DMA,
        pltpu.SemaphoreType.DMA,
    ),
  )


_BLK = 2000


def _pack16(x_lo, x_hi):
    il = lax.bitcast_convert_type(x_lo, jnp.int32) + 32768
    ih = lax.bitcast_convert_type(x_hi, jnp.int32) + 32768
    return ((il >> 16) & 0xFFFF) | (ih & -65536)


def _tc1_body(h_ref, w_ref, b_ref, a_ref, bt_ref, hphi_ref):
    hb = h_ref[...]
    t = jnp.dot(hb, w_ref[...],
                preferred_element_type=jnp.float32) + b_ref[...]
    s = hb[:, :_SD]
    qcol = jnp.exp(-t[:, 512:513])
    acat = jnp.concatenate([t[:, 0:256], s,
                            jnp.zeros((_BLK, 26), jnp.float32)], axis=1)
    bcat = jnp.concatenate([hb, t[:, 256:384], s,
                            jnp.zeros((_BLK, 10), jnp.float32), qcol,
                            jnp.zeros((_BLK, 15), jnp.float32)], axis=1)
    a_ref[...] = jnp.concatenate(
        [_pack16(acat[:, 32 * k:32 * k + 16], acat[:, 32 * k + 16:32 * k + 32])
         for k in range(9)], axis=1)
    bt_ref[...] = jnp.concatenate(
        [_pack16(bcat[:, 32 * k:32 * k + 16], bcat[:, 32 * k + 16:32 * k + 32])
         for k in range(9)], axis=1)
    hphi_ref[...] = t[:, 384:512]


_tc1 = pl.pallas_call(
    _tc1_body,
    grid=(_N // _BLK,),
    in_specs=[
        pl.BlockSpec((_BLK, _D), lambda i: (i, 0)),
        pl.BlockSpec((_D, 513), lambda i: (0, 0)),
        pl.BlockSpec((1, 513), lambda i: (0, 0)),
    ],
    out_specs=[
        pl.BlockSpec((_BLK, _RP), lambda i: (i, 0)),
        pl.BlockSpec((_BLK, _RP), lambda i: (i, 0)),
        pl.BlockSpec((_BLK, _D), lambda i: (i, 0)),
    ],
    out_shape=[
        jax.ShapeDtypeStruct((_N, _RP), jnp.int32),
        jax.ShapeDtypeStruct((_N, _RP), jnp.int32),
        jax.ShapeDtypeStruct((_N, _D), jnp.float32),
    ],
)


def _tc2_body(hphi_ref, da0, da1, db0, db1, nb0, nb1, g_ref, v_ref):
    da = da0[...] + da1[...]
    db = db0[...] + db1[...]
    nb = nb0[...] + nb1[...]
    st = nb / (db + 1e-16)
    dd = -jnp.log(st + 1e-8)
    rho = 1.0 / (1.0 + jnp.exp(-(dd - 0.5)))
    u = 1.0 - rho
    v_ref[...] = 1.0 / (da + 1e-16)
    g_ref[...] = (u * hphi_ref[...]).astype(jnp.bfloat16)


_tc2 = pl.pallas_call(
    _tc2_body,
    grid=(_N // _BLK,),
    in_specs=[pl.BlockSpec((_BLK, _D), lambda i: (i, 0))]
    + [pl.BlockSpec((_BLK, 1), lambda i: (i, 0))] * 6,
    out_specs=[
        pl.BlockSpec((_BLK, _D), lambda i: (i, 0)),
        pl.BlockSpec((_BLK, 1), lambda i: (i, 0)),
    ],
    out_shape=[
        jax.ShapeDtypeStruct((_N, _D), jnp.bfloat16),
        jax.ShapeDtypeStruct((_N, 1), jnp.float32),
    ],
)


def _tc3_body(h_ref, macc0_ref, macc1_ref, v_ref, wself_ref, wa_ref,
              wstr_ref, bias_ref, lng_ref, lnb_ref, out_ref):
    hb = h_ref[...]
    m_att = v_ref[...] * (macc0_ref[0] + macc1_ref[0])
    s = hb[:, :_SD]
    pre = (jnp.dot(hb, wself_ref[...], preferred_element_type=jnp.float32)
           + jnp.dot(m_att, wa_ref[...], preferred_element_type=jnp.float32)
           + jnp.dot(s, wstr_ref[...], preferred_element_type=jnp.float32)
           + bias_ref[...])
    hn = jnp.maximum(pre, 0.0) + hb
    mu = jnp.mean(hn, axis=1, keepdims=True)
    var = jnp.mean((hn - mu) ** 2, axis=1, keepdims=True)
    out_ref[...] = ((hn - mu) * lax.rsqrt(var + 1e-5) * lng_ref[...]
                    + lnb_ref[...])


_tc3 = pl.pallas_call(
    _tc3_body,
    grid=(_N // _BLK,),
    in_specs=[
        pl.BlockSpec((_BLK, _D), lambda i: (i, 0)),
        pl.BlockSpec((1, _BLK, _D), lambda i: (0, i, 0)),
        pl.BlockSpec((1, _BLK, _D), lambda i: (1, i, 0)),
        pl.BlockSpec((_BLK, 1), lambda i: (i, 0)),
        pl.BlockSpec((_D, _D), lambda i: (0, 0)),
        pl.BlockSpec((_D, _D), lambda i: (0, 0)),
        pl.BlockSpec((_SD, _D), lambda i: (0, 0)),
        pl.BlockSpec((1, _D), lambda i: (0, 0)),
        pl.BlockSpec((1, _D), lambda i: (0, 0)),
        pl.BlockSpec((1, _D), lambda i: (0, 0)),
    ],
    out_specs=pl.BlockSpec((_BLK, _D), lambda i: (i, 0)),
    out_shape=jax.ShapeDtypeStruct((_N, _D), jnp.float32),
)


def kernel(h, edge_index, W_att, phi_w, phi_b, W_p, W_pp, fdef_w, fdef_b,
           Wself_w, Wself_b, WA_w, WA_b, Wstr_w, Wstr_b, ln_g, ln_b):
    f32 = jnp.float32
    wcat = jnp.concatenate([W_att, W_p, W_pp, phi_w, fdef_w], axis=1)
    bcat = jnp.concatenate([jnp.zeros((384,), f32), phi_b, fdef_b])[None, :]
    a_pk, b_pk, hphi = _tc1(h, wcat, bcat)

    src = edge_index[0]
    tgt = edge_index[1]
    iar = jnp.arange(_EPAD - _E, dtype=jnp.int32) % 32
    src_g = jnp.concatenate([src, iar])
    tgt_g = jnp.concatenate([tgt, iar])
    src_s = jnp.concatenate([src, _N + iar])
    tgt_s = jnp.concatenate([tgt, _N + iar])

    es, da, db, nb = _make_pass1()(a_pk, b_pk, src_g, src_s, tgt_g, tgt_s)

    gt, vp = _tc2(hphi,
                  da[0][:, None], da[1][:, None],
                  db[0][:, None], db[1][:, None],
                  nb[0][:, None], nb[1][:, None])

    gt_i32 = lax.bitcast_convert_type(gt.reshape(_N, _D // 2, 2),
                                      jnp.int32)
    macc = _make_pass2()(gt_i32, src_g, tgt_s, es)

    wa_perm = WA_w[_PERM, :]

    out = _tc3(h, macc, macc, vp,
               Wself_w, wa_perm, Wstr_w,
               (Wself_b + WA_b + Wstr_b)[None, :],
               ln_g[None, :], ln_b[None, :])
    return out


# speedup vs baseline: 1.3770x; 1.0381x over previous
import functools

import jax
import jax.numpy as jnp
from jax import lax
from jax.experimental import pallas as pl
from jax.experimental.pallas import tpu as pltpu
from jax.experimental.pallas import tpu_sc as plsc

_N = 10000
_D = 128
_SD = 6
_E = 320000
_R = 272
_RP = 144
_NC = 2
_NS = 16
_NW = _NC * _NS
_K1 = 128
_K2 = 128
_EPW = 10112
_EPAD = _NW * _EPW
_NCH1 = _EPW // _K1
_NCH2 = _EPW // _K2
_ACC = 10240
_ACCPT = _ACC // _NS
_MR = 10048
_MRPT = _MR // _NS

_PERM = sum(([32 * j + 2 * k for k in range(16)]
             + [32 * j + 2 * k + 1 for k in range(16)]
             for j in range(4)), [])


def _pass1_body(a_hbm, b_hbm, srcg_hbm, srcs_hbm, tgtg_hbm, tgts_hbm,
                es_hbm, da_out, db_out, nb_out,
                abuf, bbuf, sgb, ssb, tgb, tsb, epb, eqb, esob,
                dash, dbsh, nbsh, zb, sem_tab, sem_idx, sem_s0, sem_s1):
    cid = lax.axis_index("c")
    sid = lax.axis_index("s")
    wid = sid * _NC + cid
    ebase = wid * _EPW
    iota16 = lax.iota(jnp.int32, 16)

    def _zb(i, _):
        zb[pl.ds(i * 16, 16)] = jnp.zeros((16,), jnp.float32)
        return ()
    lax.fori_loop(0, _ACCPT // 16, _zb, (), unroll=4)
    pltpu.sync_copy(zb, dash.at[pl.ds(sid * _ACCPT, _ACCPT)])
    pltpu.sync_copy(zb, dbsh.at[pl.ds(sid * _ACCPT, _ACCPT)])
    pltpu.sync_copy(zb, nbsh.at[pl.ds(sid * _ACCPT, _ACCPT)])
    plsc.subcore_barrier()

    def idx_start(g, sync=False):
        off = ebase + g * _K1
        slot = lax.rem(g, 3)
        pairs = ((srcg_hbm, sgb), (srcs_hbm, ssb), (tgtg_hbm, tgb),
                 (tgts_hbm, tsb))
        for hbm, buf in pairs:
            if sync:
                pltpu.sync_copy(hbm.at[pl.ds(off, _K1)], buf.at[slot])
            else:
                pltpu.async_copy(hbm.at[pl.ds(off, _K1)], buf.at[slot],
                                 sem_idx)

    def idx_wait(g):
        off = ebase + g * _K1
        slot = lax.rem(g, 3)
        for hbm, buf in ((srcg_hbm, sgb), (srcs_hbm, ssb), (tgtg_hbm, tgb),
                         (tgts_hbm, tsb)):
            pltpu.make_async_copy(hbm.at[pl.ds(off, _K1)], buf.at[slot],
                                  sem_idx).wait()

    def tab_start(g):
        slot = lax.rem(g, 2)
        islot = lax.rem(g, 3)
        pltpu.async_copy(a_hbm.at[sgb.at[islot]], abuf.at[slot], sem_tab)
        pltpu.async_copy(b_hbm.at[tgb.at[islot]], bbuf.at[slot], sem_tab)

    def tab_wait(g):
        slot = lax.rem(g, 2)
        islot = lax.rem(g, 3)
        pltpu.make_async_copy(a_hbm.at[sgb.at[islot]], abuf.at[slot],
                              sem_tab).wait()
        pltpu.make_async_copy(b_hbm.at[tgb.at[islot]], bbuf.at[slot],
                              sem_tab).wait()

    def scat_start(g):
        slot = lax.rem(g, 2)
        islot = lax.rem(g, 3)
        pltpu.sync_copy(esob.at[slot], dash.at[tsb.at[islot]], add=True)
        pltpu.sync_copy(epb.at[slot], dbsh.at[ssb.at[islot]], add=True)
        pltpu.sync_copy(eqb.at[slot], nbsh.at[ssb.at[islot]], add=True)
        pltpu.async_copy(esob.at[slot], es_hbm.at[pl.ds(ebase + g * _K1,
                                                        _K1)], sem_s0)

    def es_wait(g):
        slot = lax.rem(g, 2)
        pltpu.make_async_copy(esob.at[slot],
                              es_hbm.at[pl.ds(ebase + g * _K1, _K1)],
                              sem_s0).wait()

    idx_start(0, sync=True)
    tab_start(0)
    idx_start(1)

    def gbody(g, _):
        slot = lax.rem(g, 2)
        tab_wait(g)

        @pl.when(g < _NCH1 - 1)
        def _():
            idx_wait(g + 1)
            tab_start(g + 1)

        @pl.when(g < _NCH1 - 2)
        def _():
            idx_start(g + 2)

        @pl.when(g >= 2)
        def _():
            es_wait(g - 2)

        a2 = abuf.at[slot]
        b2 = bbuf.at[slot]
        hmask = jnp.full((16,), -65536, jnp.int32)
        for grp in range(_K1 // 16):
            def ebody(i, carry):
                sv, pv, qvv = carry
                e = grp * 16 + i
                accs = jnp.zeros((16,), jnp.float32)
                accp = jnp.zeros((16,), jnp.float32)
                qs = jnp.float32(0)
                for k in range(9):
                    xa = a2[e, pl.ds(k * 16, 16)]
                    xb = b2[e, pl.ds(k * 16, 16)]
                    la = plsc.bitcast(xa << 16, jnp.float32)
                    ha = plsc.bitcast(xa & hmask, jnp.float32)
                    lb = plsc.bitcast(xb << 16, jnp.float32)
                    hb = plsc.bitcast(xb & hmask, jnp.float32)
                    prod = la * lb + ha * hb
                    if k < 4 or k == 8:
                        accs = accs + prod
                    else:
                        accp = accp + prod
                    if k == 8:
                        qs = hb[0]
                m = (iota16 == i).astype(jnp.float32)
                sv = sv + jnp.full((16,), jnp.sum(accs), jnp.float32) * m
                pv = pv + jnp.full((16,), jnp.sum(accp), jnp.float32) * m
                qvv = qvv + jnp.full((16,), qs, jnp.float32) * m
                return sv, pv, qvv
            z16 = jnp.zeros((16,), jnp.float32)
            sv, pv, qvv = lax.fori_loop(0, 16, ebody, (z16, z16, z16),
                                        unroll=2)
            es = jnp.exp(sv)
            ep = jnp.exp(pv)
            esob[slot, pl.ds(grp * 16, 16)] = es
            epb[slot, pl.ds(grp * 16, 16)] = ep
            eqb[slot, pl.ds(grp * 16, 16)] = ep * qvv

        scat_start(g)
        return ()

    lax.fori_loop(0, _NCH1, gbody, ())
    es_wait(_NCH1 - 2)
    es_wait(_NCH1 - 1)
    plsc.subcore_barrier()
    pltpu.sync_copy(dash.at[pl.ds(sid * _ACCPT, _ACCPT)],
                    da_out.at[cid, pl.ds(sid * _ACCPT, _ACCPT)])
    pltpu.sync_copy(dbsh.at[pl.ds(sid * _ACCPT, _ACCPT)],
                    db_out.at[cid, pl.ds(sid * _ACCPT, _ACCPT)])
    pltpu.sync_copy(nbsh.at[pl.ds(sid * _ACCPT, _ACCPT)],
                    nb_out.at[cid, pl.ds(sid * _ACCPT, _ACCPT)])


@functools.cache
def _make_pass1():
  return pl.kernel(
    _pass1_body,
    out_type=(jax.ShapeDtypeStruct((_EPAD,), jnp.float32),
              jax.ShapeDtypeStruct((_NC, _ACC), jnp.float32),
              jax.ShapeDtypeStruct((_NC, _ACC), jnp.float32),
              jax.ShapeDtypeStruct((_NC, _ACC), jnp.float32)),
    mesh=plsc.VectorSubcoreMesh(core_axis_name="c", subcore_axis_name="s"),
    compiler_params=pltpu.CompilerParams(use_tc_tiling_on_sc=False,
                                         needs_layout_passes=False),
    scratch_types=(
        pltpu.VMEM((2, _K1, _RP), jnp.int32),
        pltpu.VMEM((2, _K1, _RP), jnp.int32),
        pltpu.VMEM((3, _K1), jnp.int32),
        pltpu.VMEM((3, _K1), jnp.int32),
        pltpu.VMEM((3, _K1), jnp.int32),
        pltpu.VMEM((3, _K1), jnp.int32),
        pltpu.VMEM((2, _K1), jnp.float32),
        pltpu.VMEM((2, _K1), jnp.float32),
        pltpu.VMEM((2, _K1), jnp.float32),
        pltpu.VMEM_SHARED((_ACC,), jnp.float32),
        pltpu.VMEM_SHARED((_ACC,), jnp.float32),
        pltpu.VMEM_SHARED((_ACC,), jnp.float32),
        pltpu.VMEM((_ACCPT,), jnp.float32),
        pltpu.SemaphoreType.DMA,
        pltpu.SemaphoreType.DMA,
        pltpu.SemaphoreType.DMA,
        pltpu.SemaphoreType.DMA,
    ),
  )


def _pass2_body(g_hbm, srcg_hbm, tgts_hbm, es_hbm,
                macc_out,
                gibuf, rbuf, sgb, tsb, esb, msh,
                sem_tab, sem_idx):
    cid = lax.axis_index("c")
    sid = lax.axis_index("s")
    wid = sid * _NC + cid
    ebase = wid * _EPW
    def _zg(r, _):
        for j in range(_D // 16):
            rbuf[r, pl.ds(j * 16, 16)] = jnp.zeros((16,), jnp.float32)
        return ()
    lax.fori_loop(0, _K2, _zg, (), unroll=2)
    for kk in range(4):
        pltpu.sync_copy(rbuf, msh.at[pl.ds(sid * _MRPT + kk * _K2, _K2)])
    pltpu.sync_copy(rbuf.at[pl.ds(0, _MRPT - 4 * _K2)],
                    msh.at[pl.ds(sid * _MRPT + 4 * _K2, _MRPT - 4 * _K2)])
    plsc.subcore_barrier()

    def idx_start(g, sync=False):
        off = ebase + g * _K2
        slot = lax.rem(g, 3)
        for hbm, buf in ((srcg_hbm, sgb), (tgts_hbm, tsb)):
            if sync:
                pltpu.sync_copy(hbm.at[pl.ds(off, _K2)], buf.at[slot])
            else:
                pltpu.async_copy(hbm.at[pl.ds(off, _K2)], buf.at[slot],
                                 sem_idx)

    def idx_wait(g):
        off = ebase + g * _K2
        slot = lax.rem(g, 3)
        for hbm, buf in ((srcg_hbm, sgb), (tgts_hbm, tsb)):
            pltpu.make_async_copy(hbm.at[pl.ds(off, _K2)], buf.at[slot],
                                  sem_idx).wait()

    def tab_start(g):
        slot = lax.rem(g, 3)
        off = ebase + g * _K2
        pltpu.async_copy(g_hbm.at[sgb.at[slot]], gibuf.at[lax.rem(g, 2)],
                         sem_tab)
        pltpu.async_copy(es_hbm.at[pl.ds(off, _K2)], esb.at[lax.rem(g, 2)],
                         sem_tab)

    def tab_wait(g):
        slot = lax.rem(g, 3)
        off = ebase + g * _K2
        pltpu.make_async_copy(g_hbm.at[sgb.at[slot]],
                              gibuf.at[lax.rem(g, 2)], sem_tab).wait()
        pltpu.make_async_copy(es_hbm.at[pl.ds(off, _K2)],
                              esb.at[lax.rem(g, 2)], sem_tab).wait()

    idx_start(0, sync=True)
    tab_start(0)
    idx_start(1)

    hmask = jnp.full((16,), -65536, jnp.int32)

    def gbody(g, _):
        slot = lax.rem(g, 3)
        eslot = lax.rem(g, 2)
        tab_wait(g)

        @pl.when(g < _NCH2 - 1)
        def _():
            idx_wait(g + 1)
            tab_start(g + 1)

        @pl.when(g < _NCH2 - 2)
        def _():
            idx_start(g + 2)

        def egrp(gr, _):
            esv = esb[eslot, pl.ds(gr * 16, 16)]
            base = gr * 16
            for j16 in range(16):
                sc = jnp.full((16,), esv[j16], jnp.float32)
                e = base + j16
                for j in range(_D // 32):
                    x = gibuf[eslot, e, pl.ds(j * 16, 16)]
                    lo = plsc.bitcast(x << 16, jnp.float32)
                    hi = plsc.bitcast(x & hmask, jnp.float32)
                    rbuf[e, pl.ds(j * 32, 16)] = lo * sc
                    rbuf[e, pl.ds(j * 32 + 16, 16)] = hi * sc
            return ()
        lax.fori_loop(0, _K2 // 16, egrp, ())

        pltpu.sync_copy(rbuf, msh.at[tsb.at[slot]], add=True)
        return ()

    lax.fori_loop(0, _NCH2, gbody, ())
    plsc.subcore_barrier()
    pltpu.sync_copy(msh.at[pl.ds(sid * _MRPT, _MRPT)],
                    macc_out.at[cid, pl.ds(sid * _MRPT, _MRPT)])


@functools.cache
def _make_pass2():
  return pl.kernel(
    _pass2_body,
    out_type=jax.ShapeDtypeStruct((_NC, _MR, _D), jnp.float32),
    mesh=plsc.VectorSubcoreMesh(core_axis_name="c", subcore_axis_name="s"),
    compiler_params=pltpu.CompilerParams(use_tc_tiling_on_sc=False,
                                         needs_layout_passes=False),
    scratch_types=(
        pltpu.VMEM((2, _K2, _D // 2), jnp.int32),
        pltpu.VMEM((_K2, _D), jnp.float32),
        pltpu.VMEM((3, _K2), jnp.int32),
        pltpu.VMEM((3, _K2), jnp.int32),
        pltpu.VMEM((2, _K2), jnp.float32),
        pltpu.VMEM_SHARED((_MR, _D), jnp.float32),
        pltpu.SemaphoreType.DMA,
        pltpu.SemaphoreType.DMA,
    ),
  )


_BLK = 2000


def _pack16(x_lo, x_hi):
    il = lax.bitcast_convert_type(x_lo, jnp.int32) + 32768
    ih = lax.bitcast_convert_type(x_hi, jnp.int32) + 32768
    return ((il >> 16) & 0xFFFF) | (ih & -65536)


def _tc1_body(h_ref, w_ref, b_ref, a_ref, bt_ref, hphi_ref):
    hb = h_ref[...]
    t = jnp.dot(hb, w_ref[...],
                preferred_element_type=jnp.float32) + b_ref[...]
    s = hb[:, :_SD]
    qcol = jnp.exp(-t[:, 512:513])
    acat = jnp.concatenate([t[:, 0:256], s,
                            jnp.zeros((_BLK, 26), jnp.float32)], axis=1)
    bcat = jnp.concatenate([hb, t[:, 256:384], s,
                            jnp.zeros((_BLK, 10), jnp.float32), qcol,
                            jnp.zeros((_BLK, 15), jnp.float32)], axis=1)
    a_ref[...] = jnp.concatenate(
        [_pack16(acat[:, 32 * k:32 * k + 16], acat[:, 32 * k + 16:32 * k + 32])
         for k in range(9)], axis=1)
    bt_ref[...] = jnp.concatenate(
        [_pack16(bcat[:, 32 * k:32 * k + 16], bcat[:, 32 * k + 16:32 * k + 32])
         for k in range(9)], axis=1)
    hphi_ref[...] = t[:, 384:512]


_tc1 = pl.pallas_call(
    _tc1_body,
    grid=(_N // _BLK,),
    in_specs=[
        pl.BlockSpec((_BLK, _D), lambda i: (i, 0)),
        pl.BlockSpec((_D, 513), lambda i: (0, 0)),
        pl.BlockSpec((1, 513), lambda i: (0, 0)),
    ],
    out_specs=[
        pl.BlockSpec((_BLK, _RP), lambda i: (i, 0)),
        pl.BlockSpec((_BLK, _RP), lambda i: (i, 0)),
        pl.BlockSpec((_BLK, _D), lambda i: (i, 0)),
    ],
    out_shape=[
        jax.ShapeDtypeStruct((_N, _RP), jnp.int32),
        jax.ShapeDtypeStruct((_N, _RP), jnp.int32),
        jax.ShapeDtypeStruct((_N, _D), jnp.float32),
    ],
)


def _tc2_body(hphi_ref, da0, da1, db0, db1, nb0, nb1, g_ref, v_ref):
    da = da0[...] + da1[...]
    db = db0[...] + db1[...]
    nb = nb0[...] + nb1[...]
    st = nb / (db + 1e-16)
    dd = -jnp.log(st + 1e-8)
    rho = 1.0 / (1.0 + jnp.exp(-(dd - 0.5)))
    u = 1.0 - rho
    v_ref[...] = 1.0 / (da + 1e-16)
    g_ref[...] = (u * hphi_ref[...]).astype(jnp.bfloat16)


_tc2 = pl.pallas_call(
    _tc2_body,
    grid=(_N // _BLK,),
    in_specs=[pl.BlockSpec((_BLK, _D), lambda i: (i, 0))]
    + [pl.BlockSpec((_BLK, 1), lambda i: (i, 0))] * 6,
    out_specs=[
        pl.BlockSpec((_BLK, _D), lambda i: (i, 0)),
        pl.BlockSpec((_BLK, 1), lambda i: (i, 0)),
    ],
    out_shape=[
        jax.ShapeDtypeStruct((_N, _D), jnp.bfloat16),
        jax.ShapeDtypeStruct((_N, 1), jnp.float32),
    ],
)


def _tc3_body(h_ref, macc0_ref, macc1_ref, v_ref, wself_ref, wa_ref,
              wstr_ref, bias_ref, lng_ref, lnb_ref, out_ref):
    hb = h_ref[...]
    m_att = v_ref[...] * (macc0_ref[0] + macc1_ref[0])
    s = hb[:, :_SD]
    pre = (jnp.dot(hb, wself_ref[...], preferred_element_type=jnp.float32)
           + jnp.dot(m_att, wa_ref[...], preferred_element_type=jnp.float32)
           + jnp.dot(s, wstr_ref[...], preferred_element_type=jnp.float32)
           + bias_ref[...])
    hn = jnp.maximum(pre, 0.0) + hb
    mu = jnp.mean(hn, axis=1, keepdims=True)
    var = jnp.mean((hn - mu) ** 2, axis=1, keepdims=True)
    out_ref[...] = ((hn - mu) * lax.rsqrt(var + 1e-5) * lng_ref[...]
                    + lnb_ref[...])


_tc3 = pl.pallas_call(
    _tc3_body,
    grid=(_N // _BLK,),
    in_specs=[
        pl.BlockSpec((_BLK, _D), lambda i: (i, 0)),
        pl.BlockSpec((1, _BLK, _D), lambda i: (0, i, 0)),
        pl.BlockSpec((1, _BLK, _D), lambda i: (1, i, 0)),
        pl.BlockSpec((_BLK, 1), lambda i: (i, 0)),
        pl.BlockSpec((_D, _D), lambda i: (0, 0)),
        pl.BlockSpec((_D, _D), lambda i: (0, 0)),
        pl.BlockSpec((_SD, _D), lambda i: (0, 0)),
        pl.BlockSpec((1, _D), lambda i: (0, 0)),
        pl.BlockSpec((1, _D), lambda i: (0, 0)),
        pl.BlockSpec((1, _D), lambda i: (0, 0)),
    ],
    out_specs=pl.BlockSpec((_BLK, _D), lambda i: (i, 0)),
    out_shape=jax.ShapeDtypeStruct((_N, _D), jnp.float32),
)


def kernel(h, edge_index, W_att, phi_w, phi_b, W_p, W_pp, fdef_w, fdef_b,
           Wself_w, Wself_b, WA_w, WA_b, Wstr_w, Wstr_b, ln_g, ln_b):
    f32 = jnp.float32
    wcat = jnp.concatenate([W_att, W_p, W_pp, phi_w, fdef_w], axis=1)
    bcat = jnp.concatenate([jnp.zeros((384,), f32), phi_b, fdef_b])[None, :]
    a_pk, b_pk, hphi = _tc1(h, wcat, bcat)

    src = edge_index[0]
    tgt = edge_index[1]
    iar = jnp.arange(_EPAD - _E, dtype=jnp.int32) % 32
    src_g = jnp.concatenate([src, iar])
    tgt_g = jnp.concatenate([tgt, iar])
    src_s = jnp.concatenate([src, _N + iar])
    tgt_s = jnp.concatenate([tgt, _N + iar])

    es, da, db, nb = _make_pass1()(a_pk, b_pk, src_g, src_s, tgt_g, tgt_s)

    gt, vp = _tc2(hphi,
                  da[0][:, None], da[1][:, None],
                  db[0][:, None], db[1][:, None],
                  nb[0][:, None], nb[1][:, None])

    gt_i32 = lax.bitcast_convert_type(gt.reshape(_N, _D // 2, 2),
                                      jnp.int32)
    macc = _make_pass2()(gt_i32, src_g, tgt_s, es)

    wa_perm = WA_w[_PERM, :]

    out = _tc3(h, macc, macc, vp,
               Wself_w, wa_perm, Wstr_w,
               (Wself_b + WA_b + Wstr_b)[None, :],
               ln_g[None, :], ln_b[None, :])
    return out


# speedup vs baseline: 1.3794x; 1.0017x over previous
import functools

import jax
import jax.numpy as jnp
from jax import lax
from jax.experimental import pallas as pl
from jax.experimental.pallas import tpu as pltpu
from jax.experimental.pallas import tpu_sc as plsc

_N = 10000
_D = 128
_SD = 6
_E = 320000
_R = 272
_RP = 144
_NC = 2
_NS = 16
_NW = _NC * _NS
_K1 = 128
_K2 = 128
_EPW = 10112
_EPAD = _NW * _EPW
_NCH1 = _EPW // _K1
_NCH2 = _EPW // _K2
_ACC = 10240
_ACCPT = _ACC // _NS
_MR = 10048
_MRPT = _MR // _NS

_PERM = sum(([32 * j + 2 * k for k in range(16)]
             + [32 * j + 2 * k + 1 for k in range(16)]
             for j in range(4)), [])


def _pass1_body(a_hbm, b_hbm, srcg_hbm, tgtg_hbm,
                es_hbm, da_out, db_out, nb_out,
                abuf, bbuf, sgb, tgb, epb, eqb, esob,
                dash, dbsh, nbsh, zb, sem_tab, sem_idx, sem_s0,
                sem_s1):
    cid = lax.axis_index("c")
    sid = lax.axis_index("s")
    wid = sid * _NC + cid
    ebase = wid * _EPW
    iota16 = lax.iota(jnp.int32, 16)

    def _zb(i, _):
        zb[pl.ds(i * 16, 16)] = jnp.zeros((16,), jnp.float32)
        return ()
    lax.fori_loop(0, _ACCPT // 16, _zb, (), unroll=4)
    pltpu.sync_copy(zb, dash.at[pl.ds(sid * _ACCPT, _ACCPT)])
    pltpu.sync_copy(zb, dbsh.at[pl.ds(sid * _ACCPT, _ACCPT)])
    pltpu.sync_copy(zb, nbsh.at[pl.ds(sid * _ACCPT, _ACCPT)])
    plsc.subcore_barrier()

    def idx_start(g, sync=False):
        off = ebase + g * _K1
        slot = lax.rem(g, 3)
        for hbm, buf in ((srcg_hbm, sgb), (tgtg_hbm, tgb)):
            if sync:
                pltpu.sync_copy(hbm.at[pl.ds(off, _K1)], buf.at[slot])
            else:
                pltpu.async_copy(hbm.at[pl.ds(off, _K1)], buf.at[slot],
                                 sem_idx)

    def idx_wait(g):
        off = ebase + g * _K1
        slot = lax.rem(g, 3)
        for hbm, buf in ((srcg_hbm, sgb), (tgtg_hbm, tgb)):
            pltpu.make_async_copy(hbm.at[pl.ds(off, _K1)], buf.at[slot],
                                  sem_idx).wait()

    def tab_start(g):
        slot = lax.rem(g, 2)
        islot = lax.rem(g, 3)
        pltpu.async_copy(a_hbm.at[sgb.at[islot]], abuf.at[slot], sem_tab)
        pltpu.async_copy(b_hbm.at[tgb.at[islot]], bbuf.at[slot], sem_tab)

    def tab_wait(g):
        slot = lax.rem(g, 2)
        islot = lax.rem(g, 3)
        pltpu.make_async_copy(a_hbm.at[sgb.at[islot]], abuf.at[slot],
                              sem_tab).wait()
        pltpu.make_async_copy(b_hbm.at[tgb.at[islot]], bbuf.at[slot],
                              sem_tab).wait()

    def scat_start(g):
        slot = lax.rem(g, 2)
        islot = lax.rem(g, 3)
        pltpu.sync_copy(esob.at[slot], dash.at[tgb.at[islot]], add=True)
        pltpu.sync_copy(epb.at[slot], dbsh.at[sgb.at[islot]], add=True)
        pltpu.sync_copy(eqb.at[slot], nbsh.at[sgb.at[islot]], add=True)
        pltpu.async_copy(esob.at[slot], es_hbm.at[pl.ds(ebase + g * _K1,
                                                        _K1)], sem_s0)

    def es_wait(g):
        slot = lax.rem(g, 2)
        pltpu.make_async_copy(esob.at[slot],
                              es_hbm.at[pl.ds(ebase + g * _K1, _K1)],
                              sem_s0).wait()

    idx_start(0, sync=True)
    tab_start(0)
    idx_start(1)

    def gbody(g, _):
        slot = lax.rem(g, 2)
        tab_wait(g)

        @pl.when(g < _NCH1 - 1)
        def _():
            idx_wait(g + 1)
            tab_start(g + 1)

        @pl.when(g < _NCH1 - 2)
        def _():
            idx_start(g + 2)

        @pl.when(g >= 2)
        def _():
            es_wait(g - 2)

        a2 = abuf.at[slot]
        b2 = bbuf.at[slot]
        hmask = jnp.full((16,), -65536, jnp.int32)
        for grp in range(_K1 // 16):
            def ebody(i, carry):
                sv, pv, qvv = carry
                e = grp * 16 + i
                accs = jnp.zeros((16,), jnp.float32)
                accp = jnp.zeros((16,), jnp.float32)
                qs = jnp.float32(0)
                for k in range(9):
                    xa = a2[e, pl.ds(k * 16, 16)]
                    xb = b2[e, pl.ds(k * 16, 16)]
                    la = plsc.bitcast(xa << 16, jnp.float32)
                    ha = plsc.bitcast(xa & hmask, jnp.float32)
                    lb = plsc.bitcast(xb << 16, jnp.float32)
                    hb = plsc.bitcast(xb & hmask, jnp.float32)
                    prod = la * lb + ha * hb
                    if k < 4 or k == 8:
                        accs = accs + prod
                    else:
                        accp = accp + prod
                    if k == 8:
                        qs = hb[0]
                m = (iota16 == i).astype(jnp.float32)
                sv = sv + jnp.full((16,), jnp.sum(accs), jnp.float32) * m
                pv = pv + jnp.full((16,), jnp.sum(accp), jnp.float32) * m
                qvv = qvv + jnp.full((16,), qs, jnp.float32) * m
                return sv, pv, qvv
            z16 = jnp.zeros((16,), jnp.float32)
            sv, pv, qvv = lax.fori_loop(0, 16, ebody, (z16, z16, z16),
                                        unroll=2)
            gid = ebase + g * _K1 + grp * 16 + iota16
            valid = (gid < _E).astype(jnp.float32)
            es = jnp.exp(sv) * valid
            ep = jnp.exp(pv) * valid
            esob[slot, pl.ds(grp * 16, 16)] = es
            epb[slot, pl.ds(grp * 16, 16)] = ep
            eqb[slot, pl.ds(grp * 16, 16)] = ep * qvv

        scat_start(g)
        return ()

    lax.fori_loop(0, _NCH1, gbody, ())
    es_wait(_NCH1 - 2)
    es_wait(_NCH1 - 1)
    plsc.subcore_barrier()
    pltpu.sync_copy(dash.at[pl.ds(sid * _ACCPT, _ACCPT)],
                    da_out.at[cid, pl.ds(sid * _ACCPT, _ACCPT)])
    pltpu.sync_copy(dbsh.at[pl.ds(sid * _ACCPT, _ACCPT)],
                    db_out.at[cid, pl.ds(sid * _ACCPT, _ACCPT)])
    pltpu.sync_copy(nbsh.at[pl.ds(sid * _ACCPT, _ACCPT)],
                    nb_out.at[cid, pl.ds(sid * _ACCPT, _ACCPT)])


@functools.cache
def _make_pass1():
  return pl.kernel(
    _pass1_body,
    out_type=(jax.ShapeDtypeStruct((_EPAD,), jnp.float32),
              jax.ShapeDtypeStruct((_NC, _ACC), jnp.float32),
              jax.ShapeDtypeStruct((_NC, _ACC), jnp.float32),
              jax.ShapeDtypeStruct((_NC, _ACC), jnp.float32)),
    mesh=plsc.VectorSubcoreMesh(core_axis_name="c", subcore_axis_name="s"),
    compiler_params=pltpu.CompilerParams(use_tc_tiling_on_sc=False,
                                         needs_layout_passes=False),
    scratch_types=(
        pltpu.VMEM((2, _K1, _RP), jnp.int32),
        pltpu.VMEM((2, _K1, _RP), jnp.int32),
        pltpu.VMEM((3, _K1), jnp.int32),
        pltpu.VMEM((3, _K1), jnp.int32),
        pltpu.VMEM((2, _K1), jnp.float32),
        pltpu.VMEM((2, _K1), jnp.float32),
        pltpu.VMEM((2, _K1), jnp.float32),
        pltpu.VMEM_SHARED((_ACC,), jnp.float32),
        pltpu.VMEM_SHARED((_ACC,), jnp.float32),
        pltpu.VMEM_SHARED((_ACC,), jnp.float32),
        pltpu.VMEM((_ACCPT,), jnp.float32),
        pltpu.SemaphoreType.DMA,
        pltpu.SemaphoreType.DMA,
        pltpu.SemaphoreType.DMA,
        pltpu.SemaphoreType.DMA,
    ),
  )


def _pass2_body(g_hbm, srcg_hbm, tgtg_hbm, es_hbm,
                macc_out,
                gibuf, rbuf, sgb, tsb, esb, msh,
                sem_tab, sem_idx):
    cid = lax.axis_index("c")
    sid = lax.axis_index("s")
    wid = sid * _NC + cid
    ebase = wid * _EPW
    def _zg(r, _):
        for j in range(_D // 16):
            rbuf[r, pl.ds(j * 16, 16)] = jnp.zeros((16,), jnp.float32)
        return ()
    lax.fori_loop(0, _K2, _zg, (), unroll=2)
    for kk in range(4):
        pltpu.sync_copy(rbuf, msh.at[pl.ds(sid * _MRPT + kk * _K2, _K2)])
    pltpu.sync_copy(rbuf.at[pl.ds(0, _MRPT - 4 * _K2)],
                    msh.at[pl.ds(sid * _MRPT + 4 * _K2, _MRPT - 4 * _K2)])
    plsc.subcore_barrier()

    def idx_start(g, sync=False):
        off = ebase + g * _K2
        slot = lax.rem(g, 3)
        for hbm, buf in ((srcg_hbm, sgb), (tgtg_hbm, tsb)):
            if sync:
                pltpu.sync_copy(hbm.at[pl.ds(off, _K2)], buf.at[slot])
            else:
                pltpu.async_copy(hbm.at[pl.ds(off, _K2)], buf.at[slot],
                                 sem_idx)

    def idx_wait(g):
        off = ebase + g * _K2
        slot = lax.rem(g, 3)
        for hbm, buf in ((srcg_hbm, sgb), (tgtg_hbm, tsb)):
            pltpu.make_async_copy(hbm.at[pl.ds(off, _K2)], buf.at[slot],
                                  sem_idx).wait()

    def tab_start(g):
        slot = lax.rem(g, 3)
        off = ebase + g * _K2
        pltpu.async_copy(g_hbm.at[sgb.at[slot]], gibuf.at[lax.rem(g, 2)],
                         sem_tab)
        pltpu.async_copy(es_hbm.at[pl.ds(off, _K2)], esb.at[lax.rem(g, 2)],
                         sem_tab)

    def tab_wait(g):
        slot = lax.rem(g, 3)
        off = ebase + g * _K2
        pltpu.make_async_copy(g_hbm.at[sgb.at[slot]],
                              gibuf.at[lax.rem(g, 2)], sem_tab).wait()
        pltpu.make_async_copy(es_hbm.at[pl.ds(off, _K2)],
                              esb.at[lax.rem(g, 2)], sem_tab).wait()

    idx_start(0, sync=True)
    tab_start(0)
    idx_start(1)

    hmask = jnp.full((16,), -65536, jnp.int32)

    def gbody(g, _):
        slot = lax.rem(g, 3)
        eslot = lax.rem(g, 2)
        tab_wait(g)

        @pl.when(g < _NCH2 - 1)
        def _():
            idx_wait(g + 1)
            tab_start(g + 1)

        @pl.when(g < _NCH2 - 2)
        def _():
            idx_start(g + 2)

        def egrp(gr, _):
            esv = esb[eslot, pl.ds(gr * 16, 16)]
            base = gr * 16
            for j16 in range(16):
                sc = jnp.full((16,), esv[j16], jnp.float32)
                e = base + j16
                for j in range(_D // 32):
                    x = gibuf[eslot, e, pl.ds(j * 16, 16)]
                    lo = plsc.bitcast(x << 16, jnp.float32)
                    hi = plsc.bitcast(x & hmask, jnp.float32)
                    rbuf[e, pl.ds(j * 32, 16)] = lo * sc
                    rbuf[e, pl.ds(j * 32 + 16, 16)] = hi * sc
            return ()
        lax.fori_loop(0, _K2 // 16, egrp, ())

        pltpu.sync_copy(rbuf, msh.at[tsb.at[slot]], add=True)
        return ()

    lax.fori_loop(0, _NCH2, gbody, ())
    plsc.subcore_barrier()
    pltpu.sync_copy(msh.at[pl.ds(sid * _MRPT, _MRPT)],
                    macc_out.at[cid, pl.ds(sid * _MRPT, _MRPT)])


@functools.cache
def _make_pass2():
  return pl.kernel(
    _pass2_body,
    out_type=jax.ShapeDtypeStruct((_NC, _MR, _D), jnp.float32),
    mesh=plsc.VectorSubcoreMesh(core_axis_name="c", subcore_axis_name="s"),
    compiler_params=pltpu.CompilerParams(use_tc_tiling_on_sc=False,
                                         needs_layout_passes=False),
    scratch_types=(
        pltpu.VMEM((2, _K2, _D // 2), jnp.int32),
        pltpu.VMEM((_K2, _D), jnp.float32),
        pltpu.VMEM((3, _K2), jnp.int32),
        pltpu.VMEM((3, _K2), jnp.int32),
        pltpu.VMEM((2, _K2), jnp.float32),
        pltpu.VMEM_SHARED((_MR, _D), jnp.float32),
        pltpu.SemaphoreType.DMA,
        pltpu.SemaphoreType.DMA,
    ),
  )


_BLK = 2000


def _pack16(x_lo, x_hi):
    il = lax.bitcast_convert_type(x_lo, jnp.int32) + 32768
    ih = lax.bitcast_convert_type(x_hi, jnp.int32) + 32768
    return ((il >> 16) & 0xFFFF) | (ih & -65536)


def _tc1_body(h_ref, w_ref, b_ref, a_ref, bt_ref, hphi_ref):
    hb = h_ref[...]
    t = jnp.dot(hb, w_ref[...],
                preferred_element_type=jnp.float32) + b_ref[...]
    s = hb[:, :_SD]
    qcol = jnp.exp(-t[:, 512:513])
    acat = jnp.concatenate([t[:, 0:256], s,
                            jnp.zeros((_BLK, 26), jnp.float32)], axis=1)
    bcat = jnp.concatenate([hb, t[:, 256:384], s,
                            jnp.zeros((_BLK, 10), jnp.float32), qcol,
                            jnp.zeros((_BLK, 15), jnp.float32)], axis=1)
    a_ref[...] = jnp.concatenate(
        [_pack16(acat[:, 32 * k:32 * k + 16], acat[:, 32 * k + 16:32 * k + 32])
         for k in range(9)], axis=1)
    bt_ref[...] = jnp.concatenate(
        [_pack16(bcat[:, 32 * k:32 * k + 16], bcat[:, 32 * k + 16:32 * k + 32])
         for k in range(9)], axis=1)
    hphi_ref[...] = t[:, 384:512]


_tc1 = pl.pallas_call(
    _tc1_body,
    grid=(_N // _BLK,),
    in_specs=[
        pl.BlockSpec((_BLK, _D), lambda i: (i, 0)),
        pl.BlockSpec((_D, 513), lambda i: (0, 0)),
        pl.BlockSpec((1, 513), lambda i: (0, 0)),
    ],
    out_specs=[
        pl.BlockSpec((_BLK, _RP), lambda i: (i, 0)),
        pl.BlockSpec((_BLK, _RP), lambda i: (i, 0)),
        pl.BlockSpec((_BLK, _D), lambda i: (i, 0)),
    ],
    out_shape=[
        jax.ShapeDtypeStruct((_N, _RP), jnp.int32),
        jax.ShapeDtypeStruct((_N, _RP), jnp.int32),
        jax.ShapeDtypeStruct((_N, _D), jnp.float32),
    ],
)


def _tc2_body(hphi_ref, da0, da1, db0, db1, nb0, nb1, g_ref, v_ref):
    da = da0[...] + da1[...]
    db = db0[...] + db1[...]
    nb = nb0[...] + nb1[...]
    st = nb / (db + 1e-16)
    dd = -jnp.log(st + 1e-8)
    rho = 1.0 / (1.0 + jnp.exp(-(dd - 0.5)))
    u = 1.0 - rho
    v_ref[...] = 1.0 / (da + 1e-16)
    g_ref[...] = (u * hphi_ref[...]).astype(jnp.bfloat16)


_tc2 = pl.pallas_call(
    _tc2_body,
    grid=(_N // _BLK,),
    in_specs=[pl.BlockSpec((_BLK, _D), lambda i: (i, 0))]
    + [pl.BlockSpec((_BLK, 1), lambda i: (i, 0))] * 6,
    out_specs=[
        pl.BlockSpec((_BLK, _D), lambda i: (i, 0)),
        pl.BlockSpec((_BLK, 1), lambda i: (i, 0)),
    ],
    out_shape=[
        jax.ShapeDtypeStruct((_N, _D), jnp.bfloat16),
        jax.ShapeDtypeStruct((_N, 1), jnp.float32),
    ],
)


def _tc3_body(h_ref, macc0_ref, macc1_ref, v_ref, wself_ref, wa_ref,
              wstr_ref, bias_ref, lng_ref, lnb_ref, out_ref):
    hb = h_ref[...]
    m_att = v_ref[...] * (macc0_ref[0] + macc1_ref[0])
    s = hb[:, :_SD]
    pre = (jnp.dot(hb, wself_ref[...], preferred_element_type=jnp.float32)
           + jnp.dot(m_att, wa_ref[...], preferred_element_type=jnp.float32)
           + jnp.dot(s, wstr_ref[...], preferred_element_type=jnp.float32)
           + bias_ref[...])
    hn = jnp.maximum(pre, 0.0) + hb
    mu = jnp.mean(hn, axis=1, keepdims=True)
    var = jnp.mean((hn - mu) ** 2, axis=1, keepdims=True)
    out_ref[...] = ((hn - mu) * lax.rsqrt(var + 1e-5) * lng_ref[...]
                    + lnb_ref[...])


_tc3 = pl.pallas_call(
    _tc3_body,
    grid=(_N // _BLK,),
    in_specs=[
        pl.BlockSpec((_BLK, _D), lambda i: (i, 0)),
        pl.BlockSpec((1, _BLK, _D), lambda i: (0, i, 0)),
        pl.BlockSpec((1, _BLK, _D), lambda i: (1, i, 0)),
        pl.BlockSpec((_BLK, 1), lambda i: (i, 0)),
        pl.BlockSpec((_D, _D), lambda i: (0, 0)),
        pl.BlockSpec((_D, _D), lambda i: (0, 0)),
        pl.BlockSpec((_SD, _D), lambda i: (0, 0)),
        pl.BlockSpec((1, _D), lambda i: (0, 0)),
        pl.BlockSpec((1, _D), lambda i: (0, 0)),
        pl.BlockSpec((1, _D), lambda i: (0, 0)),
    ],
    out_specs=pl.BlockSpec((_BLK, _D), lambda i: (i, 0)),
    out_shape=jax.ShapeDtypeStruct((_N, _D), jnp.float32),
)


def kernel(h, edge_index, W_att, phi_w, phi_b, W_p, W_pp, fdef_w, fdef_b,
           Wself_w, Wself_b, WA_w, WA_b, Wstr_w, Wstr_b, ln_g, ln_b):
    f32 = jnp.float32
    wcat = jnp.concatenate([W_att, W_p, W_pp, phi_w, fdef_w], axis=1)
    bcat = jnp.concatenate([jnp.zeros((384,), f32), phi_b, fdef_b])[None, :]
    a_pk, b_pk, hphi = _tc1(h, wcat, bcat)

    src = edge_index[0]
    tgt = edge_index[1]
    iar = jnp.arange(_EPAD - _E, dtype=jnp.int32) % 32
    src_g = jnp.concatenate([src, iar])
    tgt_g = jnp.concatenate([tgt, iar])

    es, da, db, nb = _make_pass1()(a_pk, b_pk, src_g, tgt_g)

    gt, vp = _tc2(hphi,
                  da[0][:, None], da[1][:, None],
                  db[0][:, None], db[1][:, None],
                  nb[0][:, None], nb[1][:, None])

    gt_i32 = lax.bitcast_convert_type(gt.reshape(_N, _D // 2, 2),
                                      jnp.int32)
    macc = _make_pass2()(gt_i32, src_g, tgt_g, es)

    wa_perm = WA_w[_PERM, :]

    out = _tc3(h, macc, macc, vp,
               Wself_w, wa_perm, Wstr_w,
               (Wself_b + WA_b + Wstr_b)[None, :],
               ln_g[None, :], ln_b[None, :])
    return out


# speedup vs baseline: 1.4776x; 1.0712x over previous
import functools

import jax
import jax.numpy as jnp
from jax import lax
from jax.experimental import pallas as pl
from jax.experimental.pallas import tpu as pltpu
from jax.experimental.pallas import tpu_sc as plsc

_N = 10000
_D = 128
_SD = 6
_E = 320000
_R = 272
_RP = 144
_NC = 2
_NS = 16
_NW = _NC * _NS
_K1 = 128
_K2 = 128
_EPW = 10112
_EPAD = _NW * _EPW
_NCH1 = _EPW // _K1
_NCH2 = _EPW // _K2
_ACC = 10240
_ACCPT = _ACC // _NS
_MR = 10048
_MRPT = _MR // _NS


def _pass1_body(a_hbm, b_hbm, srcg_hbm, tgtg_hbm,
                es_hbm, da_out, db_out, nb_out,
                abuf, bbuf, sgb, tgb, epb, eqb, esob,
                dash, dbsh, nbsh, zb, sem_tab, sem_idx, sem_s0,
                sem_s1):
    cid = lax.axis_index("c")
    sid = lax.axis_index("s")
    wid = sid * _NC + cid
    ebase = wid * _EPW
    iota16 = lax.iota(jnp.int32, 16)

    def _zb(i, _):
        zb[pl.ds(i * 16, 16)] = jnp.zeros((16,), jnp.float32)
        return ()
    lax.fori_loop(0, _ACCPT // 16, _zb, (), unroll=4)
    pltpu.sync_copy(zb, dash.at[pl.ds(sid * _ACCPT, _ACCPT)])
    pltpu.sync_copy(zb, dbsh.at[pl.ds(sid * _ACCPT, _ACCPT)])
    pltpu.sync_copy(zb, nbsh.at[pl.ds(sid * _ACCPT, _ACCPT)])
    plsc.subcore_barrier()

    def idx_start(g, sync=False):
        off = ebase + g * _K1
        slot = lax.rem(g, 3)
        for hbm, buf in ((srcg_hbm, sgb), (tgtg_hbm, tgb)):
            if sync:
                pltpu.sync_copy(hbm.at[pl.ds(off, _K1)], buf.at[slot])
            else:
                pltpu.async_copy(hbm.at[pl.ds(off, _K1)], buf.at[slot],
                                 sem_idx)

    def idx_wait(g):
        off = ebase + g * _K1
        slot = lax.rem(g, 3)
        for hbm, buf in ((srcg_hbm, sgb), (tgtg_hbm, tgb)):
            pltpu.make_async_copy(hbm.at[pl.ds(off, _K1)], buf.at[slot],
                                  sem_idx).wait()

    def tab_start(g):
        slot = lax.rem(g, 2)
        islot = lax.rem(g, 3)
        pltpu.async_copy(a_hbm.at[sgb.at[islot]], abuf.at[slot], sem_tab)
        pltpu.async_copy(b_hbm.at[tgb.at[islot]], bbuf.at[slot], sem_tab)

    def tab_wait(g):
        slot = lax.rem(g, 2)
        islot = lax.rem(g, 3)
        pltpu.make_async_copy(a_hbm.at[sgb.at[islot]], abuf.at[slot],
                              sem_tab).wait()
        pltpu.make_async_copy(b_hbm.at[tgb.at[islot]], bbuf.at[slot],
                              sem_tab).wait()

    def scat_start(g):
        slot = lax.rem(g, 2)
        islot = lax.rem(g, 3)
        pltpu.sync_copy(esob.at[slot], dash.at[tgb.at[islot]], add=True)
        pltpu.sync_copy(epb.at[slot], dbsh.at[sgb.at[islot]], add=True)
        pltpu.sync_copy(eqb.at[slot], nbsh.at[sgb.at[islot]], add=True)
        pltpu.async_copy(esob.at[slot], es_hbm.at[pl.ds(ebase + g * _K1,
                                                        _K1)], sem_s0)

    def es_wait(g):
        slot = lax.rem(g, 2)
        pltpu.make_async_copy(esob.at[slot],
                              es_hbm.at[pl.ds(ebase + g * _K1, _K1)],
                              sem_s0).wait()

    idx_start(0, sync=True)
    tab_start(0)
    idx_start(1)

    def gbody(g, _):
        slot = lax.rem(g, 2)
        tab_wait(g)

        @pl.when(g < _NCH1 - 1)
        def _():
            idx_wait(g + 1)
            tab_start(g + 1)

        @pl.when(g < _NCH1 - 2)
        def _():
            idx_start(g + 2)

        @pl.when(g >= 2)
        def _():
            es_wait(g - 2)

        a2 = abuf.at[slot]
        b2 = bbuf.at[slot]
        hmask = jnp.full((16,), -65536, jnp.int32)
        for grp in range(_K1 // 16):
            def ebody(i, carry):
                sv, pv, qvv = carry
                e = grp * 16 + i
                accs = jnp.zeros((16,), jnp.float32)
                accp = jnp.zeros((16,), jnp.float32)
                qs = jnp.float32(0)
                for k in range(9):
                    xa = a2[e, pl.ds(k * 16, 16)]
                    xb = b2[e, pl.ds(k * 16, 16)]
                    la = plsc.bitcast(xa << 16, jnp.float32)
                    ha = plsc.bitcast(xa & hmask, jnp.float32)
                    lb = plsc.bitcast(xb << 16, jnp.float32)
                    hb = plsc.bitcast(xb & hmask, jnp.float32)
                    prod = la * lb + ha * hb
                    if k < 4 or k == 8:
                        accs = accs + prod
                    else:
                        accp = accp + prod
                    if k == 8:
                        qs = hb[0]
                m = (iota16 == i).astype(jnp.float32)
                sv = sv + jnp.full((16,), jnp.sum(accs), jnp.float32) * m
                pv = pv + jnp.full((16,), jnp.sum(accp), jnp.float32) * m
                qvv = qvv + jnp.full((16,), qs, jnp.float32) * m
                return sv, pv, qvv
            z16 = jnp.zeros((16,), jnp.float32)
            sv, pv, qvv = lax.fori_loop(0, 16, ebody, (z16, z16, z16),
                                        unroll=2)
            gid = ebase + g * _K1 + grp * 16 + iota16
            valid = (gid < _E).astype(jnp.float32)
            es = jnp.exp(sv) * valid
            ep = jnp.exp(pv) * valid
            esob[slot, pl.ds(grp * 16, 16)] = es
            epb[slot, pl.ds(grp * 16, 16)] = ep
            eqb[slot, pl.ds(grp * 16, 16)] = ep * qvv

        scat_start(g)
        return ()

    lax.fori_loop(0, _NCH1, gbody, ())
    es_wait(_NCH1 - 2)
    es_wait(_NCH1 - 1)
    plsc.subcore_barrier()
    pltpu.sync_copy(dash.at[pl.ds(sid * _ACCPT, _ACCPT)],
                    da_out.at[cid, pl.ds(sid * _ACCPT, _ACCPT)])
    pltpu.sync_copy(dbsh.at[pl.ds(sid * _ACCPT, _ACCPT)],
                    db_out.at[cid, pl.ds(sid * _ACCPT, _ACCPT)])
    pltpu.sync_copy(nbsh.at[pl.ds(sid * _ACCPT, _ACCPT)],
                    nb_out.at[cid, pl.ds(sid * _ACCPT, _ACCPT)])


@functools.cache
def _make_pass1():
  return pl.kernel(
    _pass1_body,
    out_type=(jax.ShapeDtypeStruct((_EPAD,), jnp.float32),
              jax.ShapeDtypeStruct((_NC, _ACC), jnp.float32),
              jax.ShapeDtypeStruct((_NC, _ACC), jnp.float32),
              jax.ShapeDtypeStruct((_NC, _ACC), jnp.float32)),
    mesh=plsc.VectorSubcoreMesh(core_axis_name="c", subcore_axis_name="s"),
    compiler_params=pltpu.CompilerParams(use_tc_tiling_on_sc=False,
                                         needs_layout_passes=False),
    scratch_types=(
        pltpu.VMEM((2, _K1, _RP), jnp.int32),
        pltpu.VMEM((2, _K1, _RP), jnp.int32),
        pltpu.VMEM((3, _K1), jnp.int32),
        pltpu.VMEM((3, _K1), jnp.int32),
        pltpu.VMEM((2, _K1), jnp.float32),
        pltpu.VMEM((2, _K1), jnp.float32),
        pltpu.VMEM((2, _K1), jnp.float32),
        pltpu.VMEM_SHARED((_ACC,), jnp.float32),
        pltpu.VMEM_SHARED((_ACC,), jnp.float32),
        pltpu.VMEM_SHARED((_ACC,), jnp.float32),
        pltpu.VMEM((_ACCPT,), jnp.float32),
        pltpu.SemaphoreType.DMA,
        pltpu.SemaphoreType.DMA,
        pltpu.SemaphoreType.DMA,
        pltpu.SemaphoreType.DMA,
    ),
  )


def _pass2_body(g_hbm, srcg_hbm, tgtg_hbm, es_hbm, u_hbm,
                macc_out,
                gibuf, rbuf, sgb, tsb, esb, ubuf, msh,
                sem_tab, sem_idx):
    cid = lax.axis_index("c")
    sid = lax.axis_index("s")
    wid = sid * _NC + cid
    ebase = wid * _EPW

    pltpu.sync_copy(u_hbm, ubuf)
    def _zg(r, _):
        for j in range(_D // 16):
            rbuf[r, pl.ds(j * 16, 16)] = jnp.zeros((16,), jnp.float32)
        return ()
    lax.fori_loop(0, _K2, _zg, (), unroll=2)
    for kk in range(4):
        pltpu.sync_copy(rbuf, msh.at[pl.ds(sid * _MRPT + kk * _K2, _K2)])
    pltpu.sync_copy(rbuf.at[pl.ds(0, _MRPT - 4 * _K2)],
                    msh.at[pl.ds(sid * _MRPT + 4 * _K2, _MRPT - 4 * _K2)])
    plsc.subcore_barrier()

    def idx_start(g, sync=False):
        off = ebase + g * _K2
        slot = lax.rem(g, 3)
        for hbm, buf in ((srcg_hbm, sgb), (tgtg_hbm, tsb)):
            if sync:
                pltpu.sync_copy(hbm.at[pl.ds(off, _K2)], buf.at[slot])
            else:
                pltpu.async_copy(hbm.at[pl.ds(off, _K2)], buf.at[slot],
                                 sem_idx)

    def idx_wait(g):
        off = ebase + g * _K2
        slot = lax.rem(g, 3)
        for hbm, buf in ((srcg_hbm, sgb), (tgtg_hbm, tsb)):
            pltpu.make_async_copy(hbm.at[pl.ds(off, _K2)], buf.at[slot],
                                  sem_idx).wait()

    def tab_start(g):
        slot = lax.rem(g, 3)
        off = ebase + g * _K2
        pltpu.async_copy(g_hbm.at[sgb.at[slot]], gibuf.at[lax.rem(g, 2)],
                         sem_tab)
        pltpu.async_copy(es_hbm.at[pl.ds(off, _K2)], esb.at[lax.rem(g, 2)],
                         sem_tab)

    def tab_wait(g):
        slot = lax.rem(g, 3)
        off = ebase + g * _K2
        pltpu.make_async_copy(g_hbm.at[sgb.at[slot]],
                              gibuf.at[lax.rem(g, 2)], sem_tab).wait()
        pltpu.make_async_copy(es_hbm.at[pl.ds(off, _K2)],
                              esb.at[lax.rem(g, 2)], sem_tab).wait()

    idx_start(0, sync=True)
    tab_start(0)
    idx_start(1)

    hmask = jnp.full((16,), -65536, jnp.int32)

    def gbody(g, _):
        slot = lax.rem(g, 3)
        eslot = lax.rem(g, 2)
        tab_wait(g)

        @pl.when(g < _NCH2 - 1)
        def _():
            idx_wait(g + 1)
            tab_start(g + 1)

        @pl.when(g < _NCH2 - 2)
        def _():
            idx_start(g + 2)

        def egrp(gr, _):
            sidx = sgb[slot, pl.ds(gr * 16, 16)]
            uv = plsc.load_gather(ubuf, [sidx])
            esv = esb[eslot, pl.ds(gr * 16, 16)] * uv
            base = gr * 16
            for j16 in range(16):
                sc = jnp.full((16,), esv[j16], jnp.float32)
                e = base + j16
                for j in range(_D // 32):
                    x = gibuf[eslot, e, pl.ds(j * 16, 16)]
                    lo = plsc.bitcast(x << 16, jnp.float32)
                    hi = plsc.bitcast(x & hmask, jnp.float32)
                    rbuf[e, pl.ds(j * 32, 16)] = lo * sc
                    rbuf[e, pl.ds(j * 32 + 16, 16)] = hi * sc
            return ()
        lax.fori_loop(0, _K2 // 16, egrp, ())

        pltpu.sync_copy(rbuf, msh.at[tsb.at[slot]], add=True)
        return ()

    lax.fori_loop(0, _NCH2, gbody, ())
    plsc.subcore_barrier()
    pltpu.sync_copy(msh.at[pl.ds(sid * _MRPT, _MRPT)],
                    macc_out.at[cid, pl.ds(sid * _MRPT, _MRPT)])


@functools.cache
def _make_pass2():
  return pl.kernel(
    _pass2_body,
    out_type=jax.ShapeDtypeStruct((_NC, _MR, _D), jnp.float32),
    mesh=plsc.VectorSubcoreMesh(core_axis_name="c", subcore_axis_name="s"),
    compiler_params=pltpu.CompilerParams(use_tc_tiling_on_sc=False,
                                         needs_layout_passes=False),
    scratch_types=(
        pltpu.VMEM((2, _K2, _D // 2), jnp.int32),
        pltpu.VMEM((_K2, _D), jnp.float32),
        pltpu.VMEM((3, _K2), jnp.int32),
        pltpu.VMEM((3, _K2), jnp.int32),
        pltpu.VMEM((2, _K2), jnp.float32),
        pltpu.VMEM((_ACC,), jnp.float32),
        pltpu.VMEM_SHARED((_MR, _D), jnp.float32),
        pltpu.SemaphoreType.DMA,
        pltpu.SemaphoreType.DMA,
    ),
  )


_BLK = 2000


def _pack16(x_lo, x_hi):
    il = lax.bitcast_convert_type(x_lo, jnp.int32) + 32768
    ih = lax.bitcast_convert_type(x_hi, jnp.int32) + 32768
    return ((il >> 16) & 0xFFFF) | (ih & -65536)


def _tc1_body(h_ref, w_ref, b_ref, a_ref, bt_ref, hphi_ref):
    hb = h_ref[...]
    t = jnp.dot(hb, w_ref[...],
                preferred_element_type=jnp.float32) + b_ref[...]
    s = hb[:, :_SD]
    qcol = jnp.exp(-t[:, 512:513])
    acat = jnp.concatenate([t[:, 0:256], s,
                            jnp.zeros((_BLK, 26), jnp.float32)], axis=1)
    bcat = jnp.concatenate([hb, t[:, 256:384], s,
                            jnp.zeros((_BLK, 10), jnp.float32), qcol,
                            jnp.zeros((_BLK, 15), jnp.float32)], axis=1)
    a_ref[...] = jnp.concatenate(
        [_pack16(acat[:, 32 * k:32 * k + 16], acat[:, 32 * k + 16:32 * k + 32])
         for k in range(9)], axis=1)
    bt_ref[...] = jnp.concatenate(
        [_pack16(bcat[:, 32 * k:32 * k + 16], bcat[:, 32 * k + 16:32 * k + 32])
         for k in range(9)], axis=1)
    hcat = t[:, 384:512]
    hphi_ref[...] = jnp.concatenate(
        [_pack16(hcat[:, 32 * k:32 * k + 16], hcat[:, 32 * k + 16:32 * k + 32])
         for k in range(4)], axis=1)


_tc1 = pl.pallas_call(
    _tc1_body,
    grid=(_N // _BLK,),
    in_specs=[
        pl.BlockSpec((_BLK, _D), lambda i: (i, 0)),
        pl.BlockSpec((_D, 513), lambda i: (0, 0)),
        pl.BlockSpec((1, 513), lambda i: (0, 0)),
    ],
    out_specs=[
        pl.BlockSpec((_BLK, _RP), lambda i: (i, 0)),
        pl.BlockSpec((_BLK, _RP), lambda i: (i, 0)),
        pl.BlockSpec((_BLK, _D // 2), lambda i: (i, 0)),
    ],
    out_shape=[
        jax.ShapeDtypeStruct((_N, _RP), jnp.int32),
        jax.ShapeDtypeStruct((_N, _RP), jnp.int32),
        jax.ShapeDtypeStruct((_N, _D // 2), jnp.int32),
    ],
)


def _tc2u_body(db_ref, nb_ref, u_ref):
    db = db_ref[0] + db_ref[1]
    nb = nb_ref[0] + nb_ref[1]
    st = nb / (db + 1e-16)
    w = 1.6487212707001282 * (st + 1e-8)
    u_ref[...] = w / (1.0 + w)


_tc2u = pl.pallas_call(
    _tc2u_body,
    grid=(1,),
    in_specs=[
        pl.BlockSpec((2, 80, _D), lambda i: (0, 0, 0)),
        pl.BlockSpec((2, 80, _D), lambda i: (0, 0, 0)),
    ],
    out_specs=pl.BlockSpec((80, _D), lambda i: (0, 0)),
    out_shape=jax.ShapeDtypeStruct((80, _D), jnp.float32),
)


def _tc3_body(h_ref, macc0_ref, macc1_ref, da0_ref, da1_ref, wself_ref,
              wa_ref, wstr_ref, bias_ref, lng_ref, lnb_ref, out_ref):
    hb = h_ref[...]
    v = 1.0 / (da0_ref[...] + da1_ref[...] + 1e-16)
    m_att = v * (macc0_ref[0] + macc1_ref[0])
    s = hb[:, :_SD]
    pre = (jnp.dot(hb, wself_ref[...], preferred_element_type=jnp.float32)
           + jnp.dot(m_att, wa_ref[...], preferred_element_type=jnp.float32)
           + jnp.dot(s, wstr_ref[...], preferred_element_type=jnp.float32)
           + bias_ref[...])
    hn = jnp.maximum(pre, 0.0) + hb
    mu = jnp.mean(hn, axis=1, keepdims=True)
    var = jnp.mean((hn - mu) ** 2, axis=1, keepdims=True)
    out_ref[...] = ((hn - mu) * lax.rsqrt(var + 1e-5) * lng_ref[...]
                    + lnb_ref[...])


_tc3 = pl.pallas_call(
    _tc3_body,
    grid=(_N // _BLK,),
    in_specs=[
        pl.BlockSpec((_BLK, _D), lambda i: (i, 0)),
        pl.BlockSpec((1, _BLK, _D), lambda i: (0, i, 0)),
        pl.BlockSpec((1, _BLK, _D), lambda i: (1, i, 0)),
        pl.BlockSpec((_BLK, 1), lambda i: (i, 0)),
        pl.BlockSpec((_BLK, 1), lambda i: (i, 0)),
        pl.BlockSpec((_D, _D), lambda i: (0, 0)),
        pl.BlockSpec((_D, _D), lambda i: (0, 0)),
        pl.BlockSpec((_SD, _D), lambda i: (0, 0)),
        pl.BlockSpec((1, _D), lambda i: (0, 0)),
        pl.BlockSpec((1, _D), lambda i: (0, 0)),
        pl.BlockSpec((1, _D), lambda i: (0, 0)),
    ],
    out_specs=pl.BlockSpec((_BLK, _D), lambda i: (i, 0)),
    out_shape=jax.ShapeDtypeStruct((_N, _D), jnp.float32),
)


def kernel(h, edge_index, W_att, phi_w, phi_b, W_p, W_pp, fdef_w, fdef_b,
           Wself_w, Wself_b, WA_w, WA_b, Wstr_w, Wstr_b, ln_g, ln_b):
    f32 = jnp.float32
    wcat = jnp.concatenate([W_att, W_p, W_pp, phi_w, fdef_w], axis=1)
    bcat = jnp.concatenate([jnp.zeros((384,), f32), phi_b, fdef_b])[None, :]
    a_pk, b_pk, hphi = _tc1(h, wcat, bcat)

    src = edge_index[0]
    tgt = edge_index[1]
    iar = jnp.arange(_EPAD - _E, dtype=jnp.int32) % 32
    src_g = jnp.concatenate([src, iar])
    tgt_g = jnp.concatenate([tgt, iar])

    es, da, db, nb = _make_pass1()(a_pk, b_pk, src_g, tgt_g)

    u_r = _tc2u(db.reshape(2, 80, _D), nb.reshape(2, 80, _D))

    macc = _make_pass2()(hphi, src_g, tgt_g, es, u_r.reshape(_ACC))

    out = _tc3(h, macc, macc, da[0][:, None], da[1][:, None],
               Wself_w, WA_w, Wstr_w,
               (Wself_b + WA_b + Wstr_b)[None, :],
               ln_g[None, :], ln_b[None, :])
    return out
